# baseline (speedup 1.0000x reference)
import jax
import jax.numpy as jnp
from jax import lax
from jax.experimental import pallas as pl
from jax.experimental.pallas import tpu as pltpu
from jax.experimental.pallas import tpu_sc as plsc

N = 10000
E = 320000
D = 128
HID = 256
NCLS = 10
G = 64

NCORE = 2
NSUB = 16
CH = 80
BN = 1000
RSTRIPE = 624
RLAST = N - (NSUB - 1) * RSTRIPE

_f32 = jnp.float32
_i32 = jnp.int32

_SC_PARAMS = pltpu.CompilerParams(needs_layout_passes=False,
                                  use_tc_tiling_on_sc=False)


def _tc_embed(x_b, w1, ms, md, h_b, at_b, bt_b):
    h = jnp.dot(x_b[...], w1[...], preferred_element_type=_f32)
    h_b[...] = h
    at_b[...] = jnp.dot(h, ms[...], preferred_element_type=_f32)
    bt_b[...] = jnp.dot(h, md[...], preferred_element_type=_f32)


def _run_embed(x, W1, Ms, Md):
    return pl.pallas_call(
        _tc_embed,
        grid=(N // BN,),
        in_specs=[
            pl.BlockSpec((BN, D), lambda i: (i, 0)),
            pl.BlockSpec((D, 64), lambda i: (0, 0)),
            pl.BlockSpec((64, 16), lambda i: (0, 0)),
            pl.BlockSpec((64, 16), lambda i: (0, 0)),
        ],
        out_specs=[
            pl.BlockSpec((BN, 64), lambda i: (i, 0)),
            pl.BlockSpec((BN, 16), lambda i: (i, 0)),
            pl.BlockSpec((BN, 16), lambda i: (i, 0)),
        ],
        out_shape=[
            jax.ShapeDtypeStruct((N, 64), _f32),
            jax.ShapeDtypeStruct((N, 16), _f32),
            jax.ShapeDtypeStruct((N, 16), _f32),
        ],
    )(x, W1, Ms, Md)


def _striped_rows(s, do):
    @pl.when(s < NSUB - 1)
    def _():
        do(s * RSTRIPE, RSTRIPE)

    @pl.when(s == NSUB - 1)
    def _():
        do((NSUB - 1) * RSTRIPE, RLAST)


def _vcopy(src, dst, n):
    for j in range(n // 16):
        dst[pl.ds(16 * j, 16)] = src[pl.ds(16 * j, 16)]


def _sc_gat1(src_h, dst_h, at_h, bt_h, hf_h, z64, z16,
             acc0, acc1, den0, den1,
             acc_sh, den_sh,
             srcA, dstA, dstsA, arowA, browA, hrowA, svA, mA,
             srcB, dstB, dstsB, arowB, browB, hrowB, svB, mB,
             gsemA, gsemB, ssemA, ssemB):
    c = lax.axis_index("c")
    s = lax.axis_index("s")

    def _zero(off, size):
        pltpu.sync_copy(z64.at[pl.ds(0, size)], acc_sh.at[pl.ds(off, size)])
        pltpu.sync_copy(z16.at[pl.ds(0, size)], den_sh.at[pl.ds(off, size)])

    _striped_rows(s, _zero)
    plsc.subcore_barrier()

    wid = c * NSUB + s
    per_tile = E // (NCORE * NSUB)
    nchunks = per_tile // CH
    npairs = nchunks // 2
    base = wid * per_tile
    iota = lax.iota(_i32, 16)
    colsel = [2 * k + lax.shift_right_logical(iota, 3) for k in range(4)]

    bufA = (srcA, dstA, dstsA, arowA, browA, hrowA, svA, mA, gsemA, ssemA)
    bufB = (srcB, dstB, dstsB, arowB, browB, hrowB, svB, mB, gsemB, ssemB)

    def issue(i, buf):
        srcv, dstv, dsts, arow, brow, hrow, sv, m, gsem, ssem = buf
        off = base + i * CH
        pltpu.sync_copy(src_h.at[pl.ds(off, CH)], srcv)
        pltpu.sync_copy(dst_h.at[pl.ds(off, CH)], dstv)
        pltpu.async_copy(hf_h.at[srcv], hrow, gsem)
        pltpu.async_copy(at_h.at[srcv], arow, gsem)
        pltpu.async_copy(bt_h.at[dstv], brow, gsem)

    def wait_g(buf):
        srcv, dstv, dsts, arow, brow, hrow, sv, m, gsem, ssem = buf
        pltpu.make_async_copy(hf_h.at[srcv], hrow, gsem).wait()
        pltpu.make_async_copy(at_h.at[srcv], arow, gsem).wait()
        pltpu.make_async_copy(bt_h.at[dstv], brow, gsem).wait()

    def compute(buf):
        srcv, dstv, dsts, arow, brow, hrow, sv, m, gsem, ssem = buf
        _vcopy(dstv, dsts, CH)

        def edge_s(j, cc):
            for u in range(2):
                jj = 2 * j + u
                xv = arow[jj, :] + brow[jj, :]
                sv[jj, :] = jnp.exp(jnp.maximum(xv, 0.2 * xv))
            return cc

        lax.fori_loop(0, CH // 2, edge_s, 0)

        def edge_m(j, cc):
            for u in range(2):
                jj = 2 * j + u
                jv = jnp.full((16,), jj, _i32)
                for k in range(4):
                    svv = plsc.load_gather(sv, (jv, colsel[k]))
                    m[jj, pl.ds(16 * k, 16)] = (
                        hrow[jj, pl.ds(16 * k, 16)] * svv)
            return cc

        lax.fori_loop(0, CH // 2, edge_m, 0)

    def issue_s(buf):
        srcv, dstv, dsts, arow, brow, hrow, sv, m, gsem, ssem = buf
        pltpu.async_copy(sv, den_sh.at[dsts], ssem, add=True)
        pltpu.async_copy(m, acc_sh.at[dsts], ssem, add=True)

    def wait_s(buf):
        srcv, dstv, dsts, arow, brow, hrow, sv, m, gsem, ssem = buf
        pltpu.make_async_copy(sv, den_sh.at[dsts], ssem).wait()
        pltpu.make_async_copy(m, acc_sh.at[dsts], ssem).wait()

    issue(0, bufA)
    issue(1, bufB)

    def pair(g, cc):
        i0 = 2 * g
        wait_g(bufA)

        @pl.when(g > 0)
        def _():
            wait_s(bufA)

        compute(bufA)
        issue_s(bufA)

        @pl.when(i0 + 2 < nchunks)
        def _():
            issue(i0 + 2, bufA)

        wait_g(bufB)

        @pl.when(g > 0)
        def _():
            wait_s(bufB)

        compute(bufB)
        issue_s(bufB)

        @pl.when(i0 + 3 < nchunks)
        def _():
            issue(i0 + 3, bufB)

        return cc

    lax.fori_loop(0, npairs, pair, 0)
    if nchunks % 2 == 1:
        wait_g(bufA)
        wait_s(bufA)
        compute(bufA)
        issue_s(bufA)
    wait_s(bufA)
    wait_s(bufB)

    plsc.subcore_barrier()

    def _out0(off, size):
        pltpu.sync_copy(acc_sh.at[pl.ds(off, size)], acc0.at[pl.ds(off, size)])
        pltpu.sync_copy(den_sh.at[pl.ds(off, size)], den0.at[pl.ds(off, size)])

    def _out1(off, size):
        pltpu.sync_copy(acc_sh.at[pl.ds(off, size)], acc1.at[pl.ds(off, size)])
        pltpu.sync_copy(den_sh.at[pl.ds(off, size)], den1.at[pl.ds(off, size)])

    @pl.when(c == 0)
    def _():
        _striped_rows(s, _out0)

    @pl.when(c == 1)
    def _():
        _striped_rows(s, _out1)


def _run_gat1(src, dst, at, bt, hfeat, z64, z16):
    mesh = plsc.VectorSubcoreMesh(core_axis_name="c", subcore_axis_name="s",
                                  num_cores=NCORE, num_subcores=NSUB)
    buf = [
        pltpu.VMEM((CH,), _i32),
        pltpu.VMEM((CH,), _i32),
        pltpu.VMEM((CH,), _i32),
        pltpu.VMEM((CH, 16), _f32),
        pltpu.VMEM((CH, 16), _f32),
        pltpu.VMEM((CH, 64), _f32),
        pltpu.VMEM((CH, 16), _f32),
        pltpu.VMEM((CH, 64), _f32),
    ]
    fn = pl.kernel(
        _sc_gat1,
        out_type=[
            jax.ShapeDtypeStruct((N, 64), _f32),
            jax.ShapeDtypeStruct((N, 64), _f32),
            jax.ShapeDtypeStruct((N, 16), _f32),
            jax.ShapeDtypeStruct((N, 16), _f32),
        ],
        mesh=mesh,
        scratch_types=[
            pltpu.VMEM_SHARED((N, 64), _f32),
            pltpu.VMEM_SHARED((N, 16), _f32),
            *buf, *buf,
            pltpu.SemaphoreType.DMA,
            pltpu.SemaphoreType.DMA,
            pltpu.SemaphoreType.DMA,
            pltpu.SemaphoreType.DMA,
        ],
        compiler_params=_SC_PARAMS,
    )
    return fn(src, dst, at, bt, hfeat, z64, z16)


def _tc_mid(a0_b, a1_b, d0_b, d1_b, b1_r, e8, w2, a2m, h2ab_b, a2sd_b):
    acc = a0_b[...] + a1_b[...]
    den8 = (d0_b[...] + d1_b[...])[:, :8]
    denx = jnp.dot(den8, e8[...], preferred_element_type=_f32)
    h1 = jnp.maximum(acc / (denx + 1e-16) + b1_r[...], 0.0)
    h2pre = jnp.dot(h1, w2[...], preferred_element_type=_f32)
    h2ab_b[0, :, :] = h2pre[:, :128]
    h2ab_b[1, :, :] = h2pre[:, 128:]
    a2sd_b[...] = jnp.dot(h2pre, a2m[...], preferred_element_type=_f32)


def _run_mid(acc0, acc1, den0, den1, b1r, E8, W2, A2m):
    return pl.pallas_call(
        _tc_mid,
        grid=(N // BN,),
        in_specs=[
            pl.BlockSpec((BN, 64), lambda i: (i, 0)),
            pl.BlockSpec((BN, 64), lambda i: (i, 0)),
            pl.BlockSpec((BN, 16), lambda i: (i, 0)),
            pl.BlockSpec((BN, 16), lambda i: (i, 0)),
            pl.BlockSpec((1, 64), lambda i: (0, 0)),
            pl.BlockSpec((8, 64), lambda i: (0, 0)),
            pl.BlockSpec((64, HID), lambda i: (0, 0)),
            pl.BlockSpec((HID, 8), lambda i: (0, 0)),
        ],
        out_specs=[
            pl.BlockSpec((2, BN, 128), lambda i: (0, i, 0)),
            pl.BlockSpec((BN, 8), lambda i: (i, 0)),
        ],
        out_shape=[
            jax.ShapeDtypeStruct((2, N, 128), _f32),
            jax.ShapeDtypeStruct((N, 8), _f32),
        ],
    )(acc0, acc1, den0, den1, b1r, E8, W2, A2m)


def _sc_gat2(src_h, dst_h, a2_h, h2ab_h, z128, z8,
             houta, houtb, den2p, alpha_h,
             acc_sh, den_sh,
             srcA, dstA, dstsA, src2A, arowA, browA, hrowA, sbufA, s2A, mA,
             srcB, dstB, dstsB, src2B, arowB, browB, hrowB, sbufB, s2B, mB,
             gsemA, gsemB, ssemA, ssemB, dsem):
    c = lax.axis_index("c")
    s = lax.axis_index("s")

    def _zero(off, size):
        pltpu.sync_copy(z128.at[pl.ds(0, size)], acc_sh.at[pl.ds(off, size)])
        pltpu.sync_copy(z8.at[pl.ds(0, size)], den_sh.at[pl.ds(off, size)])

    _striped_rows(s, _zero)
    pltpu.sync_copy(z8.at[pl.ds(0, CH)], sbufA)
    pltpu.sync_copy(z8.at[pl.ds(0, CH)], sbufB)
    plsc.subcore_barrier()

    per_sub = E // NSUB
    nchunks = per_sub // CH
    npairs = nchunks // 2
    base = s * per_sub
    iota = lax.iota(_i32, 16)
    zi = jnp.zeros((16,), _i32)
    oi = jnp.full((16,), 1, _i32)
    coff = c * N

    bufA = (srcA, dstA, dstsA, src2A, arowA, browA, hrowA, sbufA, s2A, mA,
            gsemA, ssemA)
    bufB = (srcB, dstB, dstsB, src2B, arowB, browB, hrowB, sbufB, s2B, mB,
            gsemB, ssemB)

    def issue(i, buf):
        srcv, dstv, dsts, src2v, arow, brow, hrow, sbuf, s2v, m, gsem, ssem = buf
        off = base + i * CH
        pltpu.sync_copy(src_h.at[pl.ds(off, CH)], srcv)
        pltpu.sync_copy(dst_h.at[pl.ds(off, CH)], dstv)
        for j in range(CH // 16):
            src2v[pl.ds(16 * j, 16)] = srcv[pl.ds(16 * j, 16)] + coff
        pltpu.async_copy(h2ab_h.at[src2v], hrow, gsem)
        pltpu.async_copy(a2_h.at[srcv], arow, gsem)
        pltpu.async_copy(a2_h.at[dstv], brow, gsem)

    def wait_g(buf):
        srcv, dstv, dsts, src2v, arow, brow, hrow, sbuf, s2v, m, gsem, ssem = buf
        pltpu.make_async_copy(h2ab_h.at[src2v], hrow, gsem).wait()
        pltpu.make_async_copy(a2_h.at[srcv], arow, gsem).wait()
        pltpu.make_async_copy(a2_h.at[dstv], brow, gsem).wait()

    def compute(buf):
        srcv, dstv, dsts, src2v, arow, brow, hrow, sbuf, s2v, m, gsem, ssem = buf
        _vcopy(dstv, dsts, CH)
        for j in range(CH // 16):
            r = 16 * j + iota
            va = plsc.load_gather(arow, (r, zi))
            vb = plsc.load_gather(brow, (r, oi))
            xv = va + vb
            s16 = jnp.exp(jnp.maximum(xv, 0.2 * xv))
            plsc.store_scatter(sbuf, (r, zi), s16)
            s2v[pl.ds(16 * j, 16)] = s16

        def edge(j, cc):
            for u in range(2):
                jj = 2 * j + u
                sj = plsc.load_gather(s2v, (jnp.full((16,), jj, _i32),))
                for k in range(8):
                    m[jj, pl.ds(16 * k, 16)] = (
                        hrow[jj, pl.ds(16 * k, 16)] * sj)
            return cc

        lax.fori_loop(0, CH // 2, edge, 0)

    def issue_s(buf):
        srcv, dstv, dsts, src2v, arow, brow, hrow, sbuf, s2v, m, gsem, ssem = buf
        pltpu.async_copy(sbuf, den_sh.at[dsts], ssem, add=True)
        pltpu.async_copy(m, acc_sh.at[dsts], ssem, add=True)

    def wait_s(buf):
        srcv, dstv, dsts, src2v, arow, brow, hrow, sbuf, s2v, m, gsem, ssem = buf
        pltpu.make_async_copy(sbuf, den_sh.at[dsts], ssem).wait()
        pltpu.make_async_copy(m, acc_sh.at[dsts], ssem).wait()

    issue(0, bufA)
    issue(1, bufB)

    def pair(g, cc):
        i0 = 2 * g
        wait_g(bufA)

        @pl.when(g > 0)
        def _():
            wait_s(bufA)

        compute(bufA)
        issue_s(bufA)

        @pl.when(i0 + 2 < nchunks)
        def _():
            issue(i0 + 2, bufA)

        wait_g(bufB)

        @pl.when(g > 0)
        def _():
            wait_s(bufB)

        compute(bufB)
        issue_s(bufB)

        @pl.when(i0 + 3 < nchunks)
        def _():
            issue(i0 + 3, bufB)

        return cc

    lax.fori_loop(0, npairs, pair, 0)
    wait_s(bufA)
    wait_s(bufB)

    plsc.subcore_barrier()

    def _out0(off, size):
        pltpu.sync_copy(acc_sh.at[pl.ds(off, size)], houta.at[pl.ds(off, size)])
        pltpu.sync_copy(den_sh.at[pl.ds(off, size)], den2p.at[pl.ds(off, size)])

    def _out1(off, size):
        pltpu.sync_copy(acc_sh.at[pl.ds(off, size)], houtb.at[pl.ds(off, size)])

    @pl.when(c == 0)
    def _():
        _striped_rows(s, _out0)

    @pl.when(c == 1)
    def _():
        _striped_rows(s, _out1)

    half = nchunks // NCORE
    drow = browB
    alpha_v = s2A

    def achunk(a, cc):
        i = c * half + a
        off = base + i * CH
        pltpu.sync_copy(src_h.at[pl.ds(off, CH)], srcA)
        pltpu.sync_copy(dst_h.at[pl.ds(off, CH)], dstA)
        pltpu.async_copy(a2_h.at[srcA], arowA, gsemA)
        pltpu.async_copy(a2_h.at[dstA], browA, gsemA)
        pltpu.async_copy(den_sh.at[dstA], drow, dsem)
        pltpu.make_async_copy(a2_h.at[srcA], arowA, gsemA).wait()
        pltpu.make_async_copy(a2_h.at[dstA], browA, gsemA).wait()
        pltpu.make_async_copy(den_sh.at[dstA], drow, dsem).wait()
        for j in range(CH // 16):
            r = 16 * j + iota
            va = plsc.load_gather(arowA, (r, zi))
            vb = plsc.load_gather(browA, (r, oi))
            xv = va + vb
            s16 = jnp.exp(jnp.maximum(xv, 0.2 * xv))
            dv = plsc.load_gather(drow, (r, zi))
            alpha_v[pl.ds(16 * j, 16)] = s16 / (dv + 1e-16)
        pltpu.sync_copy(alpha_v, alpha_h.at[pl.ds(off, CH)])
        return cc

    lax.fori_loop(0, half, achunk, 0)


def _run_gat2(src, dst, a2sd, h2ab, z128, z8):
    mesh = plsc.VectorSubcoreMesh(core_axis_name="c", subcore_axis_name="s",
                                  num_cores=NCORE, num_subcores=NSUB)
    buf = [
        pltpu.VMEM((CH,), _i32),
        pltpu.VMEM((CH,), _i32),
        pltpu.VMEM((CH,), _i32),
        pltpu.VMEM((CH,), _i32),
        pltpu.VMEM((CH, 8), _f32),
        pltpu.VMEM((CH, 8), _f32),
        pltpu.VMEM((CH, 128), _f32),
        pltpu.VMEM((CH, 8), _f32),
        pltpu.VMEM((CH,), _f32),
        pltpu.VMEM((CH, 128), _f32),
    ]
    fn = pl.kernel(
        _sc_gat2,
        out_type=[
            jax.ShapeDtypeStruct((N, 128), _f32),
            jax.ShapeDtypeStruct((N, 128), _f32),
            jax.ShapeDtypeStruct((N, 8), _f32),
            jax.ShapeDtypeStruct((E,), _f32),
        ],
        mesh=mesh,
        scratch_types=[
            pltpu.VMEM_SHARED((N, 128), _f32),
            pltpu.VMEM_SHARED((N, 8), _f32),
            *buf, *buf,
            pltpu.SemaphoreType.DMA,
            pltpu.SemaphoreType.DMA,
            pltpu.SemaphoreType.DMA,
            pltpu.SemaphoreType.DMA,
            pltpu.SemaphoreType.DMA,
        ],
        compiler_params=_SC_PARAMS,
    )
    return fn(src, dst, a2sd, h2ab, z128, z8)


def _tc_head(ha_b, hb_b, den_b, bt_b, b2a, b2b, f1w, f1b, f2w, f2b,
             f3w, f3b, out_ref, pool_acc, cnt_acc):
    i = pl.program_id(0)

    @pl.when(i == 0)
    def _():
        pool_acc[...] = jnp.zeros_like(pool_acc)
        cnt_acc[...] = jnp.zeros_like(cnt_acc)

    den = den_b[...][:, :1]
    h2a = ha_b[...] / (den + 1e-16) + b2a[...]
    h2b = hb_b[...] / (den + 1e-16) + b2b[...]
    bt = bt_b[...].reshape(1, BN)
    g_iota = lax.broadcasted_iota(_i32, (G, BN), 0)
    oh = (g_iota == bt).astype(_f32)
    pool_acc[:, :128] += jnp.dot(oh, h2a, preferred_element_type=_f32)
    pool_acc[:, 128:] += jnp.dot(oh, h2b, preferred_element_type=_f32)
    cnt_acc[...] += jnp.dot(oh, jnp.ones((BN, 128), _f32),
                            preferred_element_type=_f32)

    @pl.when(i == (N // BN) - 1)
    def _():
        pooled = pool_acc[...] / jnp.maximum(cnt_acc[:, :1], 1.0)
        z = jnp.maximum(jnp.dot(pooled, f1w[...],
                                preferred_element_type=_f32) + f1b[...], 0.0)
        z = jnp.maximum(jnp.dot(z, f2w[...],
                                preferred_element_type=_f32) + f2b[...], 0.0)
        out_ref[...] = jnp.dot(z, f3w[...],
                               preferred_element_type=_f32) + f3b[...]


def _run_head(houta, houtb, den2p, batch3, b2a, b2b,
              fc1_w, fc1_b, fc2_w, fc2_b, fc3_w, fc3_b):
    return pl.pallas_call(
        _tc_head,
        grid=(N // BN,),
        in_specs=[
            pl.BlockSpec((BN, 128), lambda i: (i, 0)),
            pl.BlockSpec((BN, 128), lambda i: (i, 0)),
            pl.BlockSpec((BN, 8), lambda i: (i, 0)),
            pl.BlockSpec((1, 1, BN), lambda i: (i, 0, 0)),
            pl.BlockSpec((1, 128), lambda i: (0, 0)),
            pl.BlockSpec((1, 128), lambda i: (0, 0)),
            pl.BlockSpec((HID, 128), lambda i: (0, 0)),
            pl.BlockSpec((1, 128), lambda i: (0, 0)),
            pl.BlockSpec((128, 64), lambda i: (0, 0)),
            pl.BlockSpec((1, 64), lambda i: (0, 0)),
            pl.BlockSpec((64, NCLS), lambda i: (0, 0)),
            pl.BlockSpec((1, NCLS), lambda i: (0, 0)),
        ],
        out_specs=pl.BlockSpec((G, NCLS), lambda i: (0, 0)),
        out_shape=jax.ShapeDtypeStruct((G, NCLS), _f32),
        scratch_shapes=[
            pltpu.VMEM((G, HID), _f32),
            pltpu.VMEM((G, 128), _f32),
        ],
    )(houta, houtb, den2p, batch3, b2a, b2b,
      fc1_w, fc1_b, fc2_w, fc2_b, fc3_w, fc3_b)


def kernel(x, W1, a1_src, a1_dst, b1, W2, a2_src, a2_dst, b2,
           fc1_w, fc1_b, fc2_w, fc2_b, fc3_w, fc3_b, edge_index, batch):
    src = edge_index[0].astype(_i32)
    dst = edge_index[1].astype(_i32)

    hh = jnp.arange(64) // 8
    a1f_s = a1_src.reshape(64)
    a1f_d = a1_dst.reshape(64)
    eye8 = jnp.eye(8, dtype=_f32)
    ms_half = eye8[hh] * a1f_s[:, None]
    md_half = eye8[hh] * a1f_d[:, None]
    Ms = jnp.concatenate([ms_half, ms_half], axis=1)
    Md = jnp.concatenate([md_half, md_half], axis=1)
    E8 = jnp.repeat(eye8, 8, axis=1)
    A2m = jnp.zeros((HID, 8), _f32)
    A2m = A2m.at[:, 0].set(a2_src[0]).at[:, 1].set(a2_dst[0])

    z64 = jnp.zeros((RLAST, 64), _f32)
    z16 = jnp.zeros((RLAST, 16), _f32)
    z128 = jnp.zeros((RLAST, 128), _f32)
    z8 = jnp.zeros((RLAST, 8), _f32)

    hfeat, at, bt = _run_embed(x, W1, Ms, Md)
    acc0, acc1, den0, den1 = _run_gat1(src, dst, at, bt, hfeat, z64, z16)
    h2ab3, a2sd = _run_mid(acc0, acc1, den0, den1,
                           b1.reshape(1, 64), E8, W2, A2m)
    h2ab = h2ab3.reshape(2 * N, 128)
    houta, houtb, den2p, alpha = _run_gat2(src, dst, a2sd, h2ab, z128, z8)

    batch3 = batch.astype(_i32).reshape(N // BN, 1, BN)
    logits = _run_head(houta, houtb, den2p, batch3,
                       b2[:128].reshape(1, 128), b2[128:].reshape(1, 128),
                       fc1_w, fc1_b.reshape(1, 128),
                       fc2_w, fc2_b.reshape(1, 64),
                       fc3_w, fc3_b.reshape(1, NCLS))
    return logits, alpha.reshape(E, 1)


# speedup vs baseline: 1.3164x; 1.3164x over previous
import jax
import jax.numpy as jnp
from jax import lax
from jax.experimental import pallas as pl
from jax.experimental.pallas import tpu as pltpu
from jax.experimental.pallas import tpu_sc as plsc

N = 10000
E = 320000
D = 128
HID = 256
NCLS = 10
G = 64

NCORE = 2
NSUB = 16
CH = 80
BN = 1000
RSTRIPE = 624
RLAST = N - (NSUB - 1) * RSTRIPE

_f32 = jnp.float32
_i32 = jnp.int32

_SC_PARAMS = pltpu.CompilerParams(needs_layout_passes=False,
                                  use_tc_tiling_on_sc=False)


def _tc_embed(x_b, w1, ms, md, h_b, at_b, bt_b):
    h = jnp.dot(x_b[...], w1[...], preferred_element_type=_f32)
    h_b[...] = h
    at_b[...] = jnp.dot(h, ms[...], preferred_element_type=_f32)
    bt_b[...] = jnp.dot(h, md[...], preferred_element_type=_f32)


def _run_embed(x, W1, Ms, Md):
    return pl.pallas_call(
        _tc_embed,
        grid=(N // BN,),
        in_specs=[
            pl.BlockSpec((BN, D), lambda i: (i, 0)),
            pl.BlockSpec((D, 64), lambda i: (0, 0)),
            pl.BlockSpec((64, 16), lambda i: (0, 0)),
            pl.BlockSpec((64, 16), lambda i: (0, 0)),
        ],
        out_specs=[
            pl.BlockSpec((BN, 64), lambda i: (i, 0)),
            pl.BlockSpec((BN, 16), lambda i: (i, 0)),
            pl.BlockSpec((BN, 16), lambda i: (i, 0)),
        ],
        out_shape=[
            jax.ShapeDtypeStruct((N, 64), _f32),
            jax.ShapeDtypeStruct((N, 16), _f32),
            jax.ShapeDtypeStruct((N, 16), _f32),
        ],
    )(x, W1, Ms, Md)


def _striped_rows(s, do):
    @pl.when(s < NSUB - 1)
    def _():
        do(s * RSTRIPE, RSTRIPE)

    @pl.when(s == NSUB - 1)
    def _():
        do((NSUB - 1) * RSTRIPE, RLAST)


def _vcopy(src, dst, n):
    for j in range(n // 16):
        dst[pl.ds(16 * j, 16)] = src[pl.ds(16 * j, 16)]


def _sc_gat1(src_h, dst_h, at_h, bt_h, hf_h, z64, z16,
             acc0, acc1, den0, den1,
             acc_sh, den_sh,
             srcA, dstA, dstsA, arowA, browA, hrowA, svA, mA,
             srcB, dstB, dstsB, arowB, browB, hrowB, svB, mB,
             gsemA, gsemB, ssemA, ssemB):
    c = lax.axis_index("c")
    s = lax.axis_index("s")

    def _zero(off, size):
        pltpu.sync_copy(z64.at[pl.ds(0, size)], acc_sh.at[pl.ds(off, size)])
        pltpu.sync_copy(z16.at[pl.ds(0, size)], den_sh.at[pl.ds(off, size)])

    _striped_rows(s, _zero)
    plsc.subcore_barrier()

    wid = c * NSUB + s
    per_tile = E // (NCORE * NSUB)
    nchunks = per_tile // CH
    npairs = nchunks // 2
    base = wid * per_tile
    iota = lax.iota(_i32, 16)
    colsel = [2 * k + lax.shift_right_logical(iota, 3) for k in range(4)]

    bufA = (srcA, dstA, dstsA, arowA, browA, hrowA, svA, mA, gsemA, ssemA)
    bufB = (srcB, dstB, dstsB, arowB, browB, hrowB, svB, mB, gsemB, ssemB)

    def issue(i, buf):
        srcv, dstv, dsts, arow, brow, hrow, sv, m, gsem, ssem = buf
        off = base + i * CH
        pltpu.sync_copy(src_h.at[pl.ds(off, CH)], srcv)
        pltpu.sync_copy(dst_h.at[pl.ds(off, CH)], dstv)
        pltpu.async_copy(hf_h.at[srcv], hrow, gsem)
        pltpu.async_copy(at_h.at[srcv], arow, gsem)
        pltpu.async_copy(bt_h.at[dstv], brow, gsem)

    def wait_g(buf):
        srcv, dstv, dsts, arow, brow, hrow, sv, m, gsem, ssem = buf
        pltpu.make_async_copy(hf_h.at[srcv], hrow, gsem).wait()
        pltpu.make_async_copy(at_h.at[srcv], arow, gsem).wait()
        pltpu.make_async_copy(bt_h.at[dstv], brow, gsem).wait()

    def compute(buf):
        srcv, dstv, dsts, arow, brow, hrow, sv, m, gsem, ssem = buf
        _vcopy(dstv, dsts, CH)

        def edge_s(j, cc):
            for u in range(2):
                jj = 2 * j + u
                xv = arow[jj, :] + brow[jj, :]
                sv[jj, :] = jnp.exp(jnp.maximum(xv, 0.2 * xv))
            return cc

        lax.fori_loop(0, CH // 2, edge_s, 0)

        def edge_m(j, cc):
            for u in range(2):
                jj = 2 * j + u
                jv = jnp.full((16,), jj, _i32)
                for k in range(4):
                    svv = plsc.load_gather(sv, (jv, colsel[k]))
                    m[jj, pl.ds(16 * k, 16)] = (
                        hrow[jj, pl.ds(16 * k, 16)] * svv)
            return cc

        lax.fori_loop(0, CH // 2, edge_m, 0)

    def issue_s(buf):
        srcv, dstv, dsts, arow, brow, hrow, sv, m, gsem, ssem = buf
        pltpu.async_copy(sv, den_sh.at[dsts], ssem, add=True)
        pltpu.async_copy(m, acc_sh.at[dsts], ssem, add=True)

    def wait_s(buf):
        srcv, dstv, dsts, arow, brow, hrow, sv, m, gsem, ssem = buf
        pltpu.make_async_copy(sv, den_sh.at[dsts], ssem).wait()
        pltpu.make_async_copy(m, acc_sh.at[dsts], ssem).wait()

    issue(0, bufA)
    issue(1, bufB)

    def pair(g, cc):
        i0 = 2 * g
        wait_g(bufA)

        @pl.when(g > 0)
        def _():
            wait_s(bufA)

        compute(bufA)
        issue_s(bufA)

        @pl.when(i0 + 2 < nchunks)
        def _():
            issue(i0 + 2, bufA)

        wait_g(bufB)

        @pl.when(g > 0)
        def _():
            wait_s(bufB)

        compute(bufB)
        issue_s(bufB)

        @pl.when(i0 + 3 < nchunks)
        def _():
            issue(i0 + 3, bufB)

        return cc

    lax.fori_loop(0, npairs, pair, 0)
    if nchunks % 2 == 1:
        wait_g(bufA)
        wait_s(bufA)
        compute(bufA)
        issue_s(bufA)
    wait_s(bufA)
    wait_s(bufB)

    plsc.subcore_barrier()

    def _out0(off, size):
        pltpu.sync_copy(acc_sh.at[pl.ds(off, size)], acc0.at[pl.ds(off, size)])
        pltpu.sync_copy(den_sh.at[pl.ds(off, size)], den0.at[pl.ds(off, size)])

    def _out1(off, size):
        pltpu.sync_copy(acc_sh.at[pl.ds(off, size)], acc1.at[pl.ds(off, size)])
        pltpu.sync_copy(den_sh.at[pl.ds(off, size)], den1.at[pl.ds(off, size)])

    @pl.when(c == 0)
    def _():
        _striped_rows(s, _out0)

    @pl.when(c == 1)
    def _():
        _striped_rows(s, _out1)


def _run_gat1(src, dst, at, bt, hfeat, z64, z16):
    mesh = plsc.VectorSubcoreMesh(core_axis_name="c", subcore_axis_name="s",
                                  num_cores=NCORE, num_subcores=NSUB)
    buf = [
        pltpu.VMEM((CH,), _i32),
        pltpu.VMEM((CH,), _i32),
        pltpu.VMEM((CH,), _i32),
        pltpu.VMEM((CH, 16), _f32),
        pltpu.VMEM((CH, 16), _f32),
        pltpu.VMEM((CH, 64), _f32),
        pltpu.VMEM((CH, 16), _f32),
        pltpu.VMEM((CH, 64), _f32),
    ]
    fn = pl.kernel(
        _sc_gat1,
        out_type=[
            jax.ShapeDtypeStruct((N, 64), _f32),
            jax.ShapeDtypeStruct((N, 64), _f32),
            jax.ShapeDtypeStruct((N, 16), _f32),
            jax.ShapeDtypeStruct((N, 16), _f32),
        ],
        mesh=mesh,
        scratch_types=[
            pltpu.VMEM_SHARED((N, 64), _f32),
            pltpu.VMEM_SHARED((N, 16), _f32),
            *buf, *buf,
            pltpu.SemaphoreType.DMA,
            pltpu.SemaphoreType.DMA,
            pltpu.SemaphoreType.DMA,
            pltpu.SemaphoreType.DMA,
        ],
        compiler_params=_SC_PARAMS,
    )
    return fn(src, dst, at, bt, hfeat, z64, z16)


def _tc_mid(a0_b, a1_b, d0_b, d1_b, b1_r, e8, w2, a2m, h2ab_b, a2sd_b):
    acc = a0_b[...] + a1_b[...]
    den8 = (d0_b[...] + d1_b[...])[:, :8]
    denx = jnp.dot(den8, e8[...], preferred_element_type=_f32)
    h1 = jnp.maximum(acc / (denx + 1e-16) + b1_r[...], 0.0)
    h2pre = jnp.dot(h1, w2[...], preferred_element_type=_f32)
    h2ab_b[0, :, :] = h2pre[:, :128]
    h2ab_b[1, :, :] = h2pre[:, 128:]
    a2sd_b[...] = jnp.dot(h2pre, a2m[...], preferred_element_type=_f32)


def _run_mid(acc0, acc1, den0, den1, b1r, E8, W2, A2m):
    return pl.pallas_call(
        _tc_mid,
        grid=(N // BN,),
        in_specs=[
            pl.BlockSpec((BN, 64), lambda i: (i, 0)),
            pl.BlockSpec((BN, 64), lambda i: (i, 0)),
            pl.BlockSpec((BN, 16), lambda i: (i, 0)),
            pl.BlockSpec((BN, 16), lambda i: (i, 0)),
            pl.BlockSpec((1, 64), lambda i: (0, 0)),
            pl.BlockSpec((8, 64), lambda i: (0, 0)),
            pl.BlockSpec((64, HID), lambda i: (0, 0)),
            pl.BlockSpec((HID, 16), lambda i: (0, 0)),
        ],
        out_specs=[
            pl.BlockSpec((2, BN, 128), lambda i: (0, i, 0)),
            pl.BlockSpec((BN, 16), lambda i: (i, 0)),
        ],
        out_shape=[
            jax.ShapeDtypeStruct((2, N, 128), _f32),
            jax.ShapeDtypeStruct((N, 16), _f32),
        ],
    )(acc0, acc1, den0, den1, b1r, E8, W2, A2m)


def _sc_gat2(src_h, dst_h, a2_h, h2ab_h, z128, z16,
             houta, houtb, den2p, alpha_h,
             acc_sh, den_sh,
             srcA, dstA, src2A, arowA, browA, hrowA, sbufA, s2A,
             srcB, dstB, src2B, arowB, browB, hrowB, sbufB, s2B,
             gsemA, gsemB, dsem):
    c = lax.axis_index("c")
    s = lax.axis_index("s")

    def _zero(off, size):
        pltpu.sync_copy(z128.at[pl.ds(0, size)], acc_sh.at[pl.ds(off, size)])
        pltpu.sync_copy(z16.at[pl.ds(0, size)], den_sh.at[pl.ds(off, size)])

    _striped_rows(s, _zero)
    pltpu.sync_copy(z16.at[pl.ds(0, CH)], sbufA)
    pltpu.sync_copy(z16.at[pl.ds(0, CH)], sbufB)
    plsc.subcore_barrier()

    per_sub = E // NSUB
    nchunks = per_sub // CH
    npairs = nchunks // 2
    base = s * per_sub
    iota = lax.iota(_i32, 16)
    zi = jnp.zeros((16,), _i32)
    oi = jnp.full((16,), 1, _i32)
    coff = c * N

    bufA = (srcA, dstA, src2A, arowA, browA, hrowA, sbufA, s2A, gsemA)
    bufB = (srcB, dstB, src2B, arowB, browB, hrowB, sbufB, s2B, gsemB)

    def issue(i, buf):
        srcv, dstv, src2v, arow, brow, hrow, sbuf, s2v, gsem = buf
        off = base + i * CH
        pltpu.sync_copy(src_h.at[pl.ds(off, CH)], srcv)
        pltpu.sync_copy(dst_h.at[pl.ds(off, CH)], dstv)
        for j in range(CH // 16):
            src2v[pl.ds(16 * j, 16)] = srcv[pl.ds(16 * j, 16)] + coff
        pltpu.async_copy(h2ab_h.at[src2v], hrow, gsem)
        pltpu.async_copy(a2_h.at[srcv], arow, gsem)
        pltpu.async_copy(a2_h.at[dstv], brow, gsem)

    def wait_g(buf):
        srcv, dstv, src2v, arow, brow, hrow, sbuf, s2v, gsem = buf
        pltpu.make_async_copy(h2ab_h.at[src2v], hrow, gsem).wait()
        pltpu.make_async_copy(a2_h.at[srcv], arow, gsem).wait()
        pltpu.make_async_copy(a2_h.at[dstv], brow, gsem).wait()

    def compute(buf):
        srcv, dstv, src2v, arow, brow, hrow, sbuf, s2v, gsem = buf
        for j in range(CH // 16):
            r = 16 * j + iota
            va = plsc.load_gather(arow, (r, zi))
            vb = plsc.load_gather(brow, (r, oi))
            xv = va + vb
            s16 = jnp.exp(jnp.maximum(xv, 0.2 * xv))
            plsc.store_scatter(sbuf, (r, zi), s16)
            s2v[pl.ds(16 * j, 16)] = s16

        def edge(j, cc):
            for u in range(2):
                jj = 2 * j + u
                sj = plsc.load_gather(s2v, (jnp.full((16,), jj, _i32),))
                for k in range(8):
                    hrow[jj, pl.ds(16 * k, 16)] = (
                        hrow[jj, pl.ds(16 * k, 16)] * sj)
            return cc

        lax.fori_loop(0, CH // 2, edge, 0)
        pltpu.sync_copy(sbuf, den_sh.at[dstv], add=True)
        pltpu.sync_copy(hrow, acc_sh.at[dstv], add=True)

    issue(0, bufA)

    def pair(g, cc):
        i0 = 2 * g
        issue(i0 + 1, bufB)
        wait_g(bufA)
        compute(bufA)

        @pl.when(i0 + 2 < nchunks)
        def _():
            issue(i0 + 2, bufA)

        wait_g(bufB)
        compute(bufB)
        return cc

    lax.fori_loop(0, npairs, pair, 0)

    plsc.subcore_barrier()

    def _out0(off, size):
        pltpu.sync_copy(acc_sh.at[pl.ds(off, size)], houta.at[pl.ds(off, size)])
        pltpu.sync_copy(den_sh.at[pl.ds(off, size)], den2p.at[pl.ds(off, size)])

    def _out1(off, size):
        pltpu.sync_copy(acc_sh.at[pl.ds(off, size)], houtb.at[pl.ds(off, size)])

    @pl.when(c == 0)
    def _():
        _striped_rows(s, _out0)

    @pl.when(c == 1)
    def _():
        _striped_rows(s, _out1)

    half = nchunks // NCORE
    drow = browB
    alpha_v = s2A

    def achunk(a, cc):
        i = c * half + a
        off = base + i * CH
        pltpu.sync_copy(src_h.at[pl.ds(off, CH)], srcA)
        pltpu.sync_copy(dst_h.at[pl.ds(off, CH)], dstA)
        pltpu.async_copy(a2_h.at[srcA], arowA, gsemA)
        pltpu.async_copy(a2_h.at[dstA], browA, gsemA)
        pltpu.async_copy(den_sh.at[dstA], drow, dsem)
        pltpu.make_async_copy(a2_h.at[srcA], arowA, gsemA).wait()
        pltpu.make_async_copy(a2_h.at[dstA], browA, gsemA).wait()
        pltpu.make_async_copy(den_sh.at[dstA], drow, dsem).wait()
        for j in range(CH // 16):
            r = 16 * j + iota
            va = plsc.load_gather(arowA, (r, zi))
            vb = plsc.load_gather(browA, (r, oi))
            xv = va + vb
            s16 = jnp.exp(jnp.maximum(xv, 0.2 * xv))
            dv = plsc.load_gather(drow, (r, zi))
            alpha_v[pl.ds(16 * j, 16)] = s16 / (dv + 1e-16)
        pltpu.sync_copy(alpha_v, alpha_h.at[pl.ds(off, CH)])
        return cc

    lax.fori_loop(0, half, achunk, 0)


def _run_gat2(src, dst, a2sd, h2ab, z128, z16):
    mesh = plsc.VectorSubcoreMesh(core_axis_name="c", subcore_axis_name="s",
                                  num_cores=NCORE, num_subcores=NSUB)
    buf = [
        pltpu.VMEM((CH,), _i32),
        pltpu.VMEM((CH,), _i32),
        pltpu.VMEM((CH,), _i32),
        pltpu.VMEM((CH, 16), _f32),
        pltpu.VMEM((CH, 16), _f32),
        pltpu.VMEM((CH, 128), _f32),
        pltpu.VMEM((CH, 16), _f32),
        pltpu.VMEM((CH,), _f32),
    ]
    fn = pl.kernel(
        _sc_gat2,
        out_type=[
            jax.ShapeDtypeStruct((N, 128), _f32),
            jax.ShapeDtypeStruct((N, 128), _f32),
            jax.ShapeDtypeStruct((N, 16), _f32),
            jax.ShapeDtypeStruct((E,), _f32),
        ],
        mesh=mesh,
        scratch_types=[
            pltpu.VMEM_SHARED((N, 128), _f32),
            pltpu.VMEM_SHARED((N, 16), _f32),
            *buf, *buf,
            pltpu.SemaphoreType.DMA,
            pltpu.SemaphoreType.DMA,
            pltpu.SemaphoreType.DMA,
        ],
        compiler_params=_SC_PARAMS,
    )
    return fn(src, dst, a2sd, h2ab, z128, z16)


def _tc_head(ha_b, hb_b, den_b, bt_b, b2a, b2b, f1w, f1b, f2w, f2b,
             f3w, f3b, out_ref, pool_acc, cnt_acc):
    i = pl.program_id(0)

    @pl.when(i == 0)
    def _():
        pool_acc[...] = jnp.zeros_like(pool_acc)
        cnt_acc[...] = jnp.zeros_like(cnt_acc)

    den = den_b[...][:, :1]
    h2a = ha_b[...] / (den + 1e-16) + b2a[...]
    h2b = hb_b[...] / (den + 1e-16) + b2b[...]
    bt = bt_b[...].reshape(1, BN)
    g_iota = lax.broadcasted_iota(_i32, (G, BN), 0)
    oh = (g_iota == bt).astype(_f32)
    pool_acc[:, :128] += jnp.dot(oh, h2a, preferred_element_type=_f32)
    pool_acc[:, 128:] += jnp.dot(oh, h2b, preferred_element_type=_f32)
    cnt_acc[...] += jnp.dot(oh, jnp.ones((BN, 128), _f32),
                            preferred_element_type=_f32)

    @pl.when(i == (N // BN) - 1)
    def _():
        pooled = pool_acc[...] / jnp.maximum(cnt_acc[:, :1], 1.0)
        z = jnp.maximum(jnp.dot(pooled, f1w[...],
                                preferred_element_type=_f32) + f1b[...], 0.0)
        z = jnp.maximum(jnp.dot(z, f2w[...],
                                preferred_element_type=_f32) + f2b[...], 0.0)
        out_ref[...] = jnp.dot(z, f3w[...],
                               preferred_element_type=_f32) + f3b[...]


def _run_head(houta, houtb, den2p, batch3, b2a, b2b,
              fc1_w, fc1_b, fc2_w, fc2_b, fc3_w, fc3_b):
    return pl.pallas_call(
        _tc_head,
        grid=(N // BN,),
        in_specs=[
            pl.BlockSpec((BN, 128), lambda i: (i, 0)),
            pl.BlockSpec((BN, 128), lambda i: (i, 0)),
            pl.BlockSpec((BN, 16), lambda i: (i, 0)),
            pl.BlockSpec((1, 1, BN), lambda i: (i, 0, 0)),
            pl.BlockSpec((1, 128), lambda i: (0, 0)),
            pl.BlockSpec((1, 128), lambda i: (0, 0)),
            pl.BlockSpec((HID, 128), lambda i: (0, 0)),
            pl.BlockSpec((1, 128), lambda i: (0, 0)),
            pl.BlockSpec((128, 64), lambda i: (0, 0)),
            pl.BlockSpec((1, 64), lambda i: (0, 0)),
            pl.BlockSpec((64, NCLS), lambda i: (0, 0)),
            pl.BlockSpec((1, NCLS), lambda i: (0, 0)),
        ],
        out_specs=pl.BlockSpec((G, NCLS), lambda i: (0, 0)),
        out_shape=jax.ShapeDtypeStruct((G, NCLS), _f32),
        scratch_shapes=[
            pltpu.VMEM((G, HID), _f32),
            pltpu.VMEM((G, 128), _f32),
        ],
    )(houta, houtb, den2p, batch3, b2a, b2b,
      fc1_w, fc1_b, fc2_w, fc2_b, fc3_w, fc3_b)


def kernel(x, W1, a1_src, a1_dst, b1, W2, a2_src, a2_dst, b2,
           fc1_w, fc1_b, fc2_w, fc2_b, fc3_w, fc3_b, edge_index, batch):
    src = edge_index[0].astype(_i32)
    dst = edge_index[1].astype(_i32)

    hh = jnp.arange(64) // 8
    a1f_s = a1_src.reshape(64)
    a1f_d = a1_dst.reshape(64)
    eye8 = jnp.eye(8, dtype=_f32)
    ms_half = eye8[hh] * a1f_s[:, None]
    md_half = eye8[hh] * a1f_d[:, None]
    Ms = jnp.concatenate([ms_half, ms_half], axis=1)
    Md = jnp.concatenate([md_half, md_half], axis=1)
    E8 = jnp.repeat(eye8, 8, axis=1)
    A2m = jnp.zeros((HID, 16), _f32)
    A2m = A2m.at[:, 0].set(a2_src[0]).at[:, 1].set(a2_dst[0])

    z64 = jnp.zeros((RLAST, 64), _f32)
    z16 = jnp.zeros((RLAST, 16), _f32)
    z128 = jnp.zeros((RLAST, 128), _f32)

    hfeat, at, bt = _run_embed(x, W1, Ms, Md)
    acc0, acc1, den0, den1 = _run_gat1(src, dst, at, bt, hfeat, z64, z16)
    h2ab3, a2sd = _run_mid(acc0, acc1, den0, den1,
                           b1.reshape(1, 64), E8, W2, A2m)
    h2ab = h2ab3.reshape(2 * N, 128)
    houta, houtb, den2p, alpha = _run_gat2(src, dst, a2sd, h2ab, z128, z16)

    batch3 = batch.astype(_i32).reshape(N // BN, 1, BN)
    logits = _run_head(houta, houtb, den2p, batch3,
                       b2[:128].reshape(1, 128), b2[128:].reshape(1, 128),
                       fc1_w, fc1_b.reshape(1, 128),
                       fc2_w, fc2_b.reshape(1, 64),
                       fc3_w, fc3_b.reshape(1, NCLS))
    return logits, alpha.reshape(E, 1)


# speedup vs baseline: 1.3917x; 1.0572x over previous
import jax
import jax.numpy as jnp
from jax import lax
from jax.experimental import pallas as pl
from jax.experimental.pallas import tpu as pltpu
from jax.experimental.pallas import tpu_sc as plsc

N = 10000
E = 320000
D = 128
HID = 256
NCLS = 10
G = 64

NCORE = 2
NSUB = 16
CH = 80
BN = 1000
RSTRIPE = 624
RLAST = N - (NSUB - 1) * RSTRIPE

_f32 = jnp.float32
_i32 = jnp.int32

_SC_PARAMS = pltpu.CompilerParams(needs_layout_passes=False,
                                  use_tc_tiling_on_sc=False)


def _tc_embed(x_b, w1, ms, md, h_b, at_b, bt_b):
    h = jnp.dot(x_b[...], w1[...], preferred_element_type=_f32)
    h_b[...] = h
    at_b[...] = jnp.dot(h, ms[...], preferred_element_type=_f32)
    bt_b[...] = jnp.dot(h, md[...], preferred_element_type=_f32)


def _run_embed(x, W1, Ms, Md):
    return pl.pallas_call(
        _tc_embed,
        grid=(N // BN,),
        in_specs=[
            pl.BlockSpec((BN, D), lambda i: (i, 0)),
            pl.BlockSpec((D, 64), lambda i: (0, 0)),
            pl.BlockSpec((64, 16), lambda i: (0, 0)),
            pl.BlockSpec((64, 16), lambda i: (0, 0)),
        ],
        out_specs=[
            pl.BlockSpec((BN, 64), lambda i: (i, 0)),
            pl.BlockSpec((BN, 16), lambda i: (i, 0)),
            pl.BlockSpec((BN, 16), lambda i: (i, 0)),
        ],
        out_shape=[
            jax.ShapeDtypeStruct((N, 64), _f32),
            jax.ShapeDtypeStruct((N, 16), _f32),
            jax.ShapeDtypeStruct((N, 16), _f32),
        ],
    )(x, W1, Ms, Md)


def _striped_rows(s, do):
    @pl.when(s < NSUB - 1)
    def _():
        do(s * RSTRIPE, RSTRIPE)

    @pl.when(s == NSUB - 1)
    def _():
        do((NSUB - 1) * RSTRIPE, RLAST)


def _vcopy(src, dst, n):
    for j in range(n // 16):
        dst[pl.ds(16 * j, 16)] = src[pl.ds(16 * j, 16)]


def _sc_gat1(ed_h, at_h, bt_h, hf_h, z64, z16,
             acc0, acc1, den0, den1,
             acc_sh, den_sh,
             edqA, srcA, dstA, dstsA, arowA, browA, hrowA, svA, mA,
             edqB, srcB, dstB, dstsB, arowB, browB, hrowB, svB, mB,
             gsemA, gsemB, ssemA, ssemB, isemA, isemB):
    c = lax.axis_index("c")
    s = lax.axis_index("s")

    def _zero(off, size):
        pltpu.sync_copy(z64.at[pl.ds(0, size)], acc_sh.at[pl.ds(off, size)])
        pltpu.sync_copy(z16.at[pl.ds(0, size)], den_sh.at[pl.ds(off, size)])

    _striped_rows(s, _zero)
    plsc.subcore_barrier()

    wid = c * NSUB + s
    per_tile = E // (NCORE * NSUB)
    nchunks = per_tile // CH
    npairs = nchunks // 2
    base = wid * per_tile
    iota = lax.iota(_i32, 16)
    colsel = [2 * k + lax.shift_right_logical(iota, 3) for k in range(4)]
    zi = jnp.zeros((16,), _i32)
    oi = jnp.full((16,), 1, _i32)

    bufA = (edqA, srcA, dstA, dstsA, arowA, browA, hrowA, svA, mA,
            gsemA, ssemA, isemA)
    bufB = (edqB, srcB, dstB, dstsB, arowB, browB, hrowB, svB, mB,
            gsemB, ssemB, isemB)

    def issue(i, buf):
        edq, srcv, dstv, dsts, arow, brow, hrow, sv, m, gsem, ssem, isem = buf
        off = base + i * CH
        pltpu.make_async_copy(ed_h.at[pl.ds(off, CH)], edq, isem).wait()
        for j in range(CH // 16):
            r = 16 * j + iota
            sv_ = plsc.load_gather(edq, (r, zi))
            dv_ = plsc.load_gather(edq, (r, oi))
            srcv[pl.ds(16 * j, 16)] = sv_
            dstv[pl.ds(16 * j, 16)] = dv_
        pltpu.async_copy(hf_h.at[srcv], hrow, gsem)
        pltpu.async_copy(at_h.at[srcv], arow, gsem)
        pltpu.async_copy(bt_h.at[dstv], brow, gsem)

        @pl.when(i + 2 < nchunks)
        def _():
            pltpu.async_copy(ed_h.at[pl.ds(off + 2 * CH, CH)], edq, isem)

    def wait_g(buf):
        edq, srcv, dstv, dsts, arow, brow, hrow, sv, m, gsem, ssem, isem = buf
        pltpu.make_async_copy(hf_h.at[srcv], hrow, gsem).wait()
        pltpu.make_async_copy(at_h.at[srcv], arow, gsem).wait()
        pltpu.make_async_copy(bt_h.at[dstv], brow, gsem).wait()

    def compute(buf):
        edq, srcv, dstv, dsts, arow, brow, hrow, sv, m, gsem, ssem, isem = buf
        _vcopy(dstv, dsts, CH)

        def edge_s(j, cc):
            for u in range(2):
                jj = 2 * j + u
                xv = arow[jj, :] + brow[jj, :]
                sv[jj, :] = jnp.exp(jnp.maximum(xv, 0.2 * xv))
            return cc

        lax.fori_loop(0, CH // 2, edge_s, 0)

        def edge_m(j, cc):
            for u in range(2):
                jj = 2 * j + u
                jv = jnp.full((16,), jj, _i32)
                for k in range(4):
                    svv = plsc.load_gather(sv, (jv, colsel[k]))
                    m[jj, pl.ds(16 * k, 16)] = (
                        hrow[jj, pl.ds(16 * k, 16)] * svv)
            return cc

        lax.fori_loop(0, CH // 2, edge_m, 0)

    def issue_s(buf):
        edq, srcv, dstv, dsts, arow, brow, hrow, sv, m, gsem, ssem, isem = buf
        pltpu.async_copy(sv, den_sh.at[dsts], ssem, add=True)
        pltpu.async_copy(m, acc_sh.at[dsts], ssem, add=True)

    def wait_s(buf):
        edq, srcv, dstv, dsts, arow, brow, hrow, sv, m, gsem, ssem, isem = buf
        pltpu.make_async_copy(sv, den_sh.at[dsts], ssem).wait()
        pltpu.make_async_copy(m, acc_sh.at[dsts], ssem).wait()

    pltpu.async_copy(ed_h.at[pl.ds(base, CH)], edqA, isemA)
    pltpu.async_copy(ed_h.at[pl.ds(base + CH, CH)], edqB, isemB)
    issue(0, bufA)
    issue(1, bufB)

    def pair(g, cc):
        i0 = 2 * g
        wait_g(bufA)

        @pl.when(g > 0)
        def _():
            wait_s(bufA)

        compute(bufA)
        issue_s(bufA)

        @pl.when(i0 + 2 < nchunks)
        def _():
            issue(i0 + 2, bufA)

        wait_g(bufB)

        @pl.when(g > 0)
        def _():
            wait_s(bufB)

        compute(bufB)
        issue_s(bufB)

        @pl.when(i0 + 3 < nchunks)
        def _():
            issue(i0 + 3, bufB)

        return cc

    lax.fori_loop(0, npairs, pair, 0)
    if nchunks % 2 == 1:
        wait_g(bufA)
        wait_s(bufA)
        compute(bufA)
        issue_s(bufA)
    wait_s(bufA)
    wait_s(bufB)

    plsc.subcore_barrier()

    def _out0(off, size):
        pltpu.sync_copy(acc_sh.at[pl.ds(off, size)], acc0.at[pl.ds(off, size)])
        pltpu.sync_copy(den_sh.at[pl.ds(off, size)], den0.at[pl.ds(off, size)])

    def _out1(off, size):
        pltpu.sync_copy(acc_sh.at[pl.ds(off, size)], acc1.at[pl.ds(off, size)])
        pltpu.sync_copy(den_sh.at[pl.ds(off, size)], den1.at[pl.ds(off, size)])

    @pl.when(c == 0)
    def _():
        _striped_rows(s, _out0)

    @pl.when(c == 1)
    def _():
        _striped_rows(s, _out1)


def _run_gat1(ed2, at, bt, hfeat, z64, z16):
    mesh = plsc.VectorSubcoreMesh(core_axis_name="c", subcore_axis_name="s",
                                  num_cores=NCORE, num_subcores=NSUB)
    buf = [
        pltpu.VMEM((CH, 2), _i32),
        pltpu.VMEM((CH,), _i32),
        pltpu.VMEM((CH,), _i32),
        pltpu.VMEM((CH,), _i32),
        pltpu.VMEM((CH, 16), _f32),
        pltpu.VMEM((CH, 16), _f32),
        pltpu.VMEM((CH, 64), _f32),
        pltpu.VMEM((CH, 16), _f32),
        pltpu.VMEM((CH, 64), _f32),
    ]
    fn = pl.kernel(
        _sc_gat1,
        out_type=[
            jax.ShapeDtypeStruct((N, 64), _f32),
            jax.ShapeDtypeStruct((N, 64), _f32),
            jax.ShapeDtypeStruct((N, 16), _f32),
            jax.ShapeDtypeStruct((N, 16), _f32),
        ],
        mesh=mesh,
        scratch_types=[
            pltpu.VMEM_SHARED((N, 64), _f32),
            pltpu.VMEM_SHARED((N, 16), _f32),
            *buf, *buf,
            pltpu.SemaphoreType.DMA,
            pltpu.SemaphoreType.DMA,
            pltpu.SemaphoreType.DMA,
            pltpu.SemaphoreType.DMA,
            pltpu.SemaphoreType.DMA,
            pltpu.SemaphoreType.DMA,
        ],
        compiler_params=_SC_PARAMS,
    )
    return fn(ed2, at, bt, hfeat, z64, z16)


def _tc_mid(a0_b, a1_b, d0_b, d1_b, b1_r, e8, w2, a2m, h2ab_b, a2sd_b):
    acc = a0_b[...] + a1_b[...]
    den8 = (d0_b[...] + d1_b[...])[:, :8]
    denx = jnp.dot(den8, e8[...], preferred_element_type=_f32)
    h1 = jnp.maximum(acc / (denx + 1e-16) + b1_r[...], 0.0)
    h2pre = jnp.dot(h1, w2[...], preferred_element_type=_f32)
    h2ab_b[0, :, :] = h2pre[:, :128]
    h2ab_b[1, :, :] = h2pre[:, 128:]
    a2sd_b[...] = jnp.dot(h2pre, a2m[...], preferred_element_type=_f32)


def _run_mid(acc0, acc1, den0, den1, b1r, E8, W2, A2m):
    return pl.pallas_call(
        _tc_mid,
        grid=(N // BN,),
        in_specs=[
            pl.BlockSpec((BN, 64), lambda i: (i, 0)),
            pl.BlockSpec((BN, 64), lambda i: (i, 0)),
            pl.BlockSpec((BN, 16), lambda i: (i, 0)),
            pl.BlockSpec((BN, 16), lambda i: (i, 0)),
            pl.BlockSpec((1, 64), lambda i: (0, 0)),
            pl.BlockSpec((8, 64), lambda i: (0, 0)),
            pl.BlockSpec((64, HID), lambda i: (0, 0)),
            pl.BlockSpec((HID, 16), lambda i: (0, 0)),
        ],
        out_specs=[
            pl.BlockSpec((2, BN, 128), lambda i: (0, i, 0)),
            pl.BlockSpec((BN, 16), lambda i: (i, 0)),
        ],
        out_shape=[
            jax.ShapeDtypeStruct((2, N, 128), _f32),
            jax.ShapeDtypeStruct((N, 16), _f32),
        ],
    )(acc0, acc1, den0, den1, b1r, E8, W2, A2m)


def _sc_gat2(ed_h, a2_h, h2ab_h, z128, z16,
             houta, houtb, den2p, alpha_h,
             acc_sh, den_sh,
             edqA, srcA, dstA, src2A, arowA, browA, hrowA, sbufA, s2A,
             edqB, srcB, dstB, src2B, arowB, browB, hrowB, sbufB, s2B,
             gsemA, gsemB, dsem, isemA, isemB):
    c = lax.axis_index("c")
    s = lax.axis_index("s")

    def _zero(off, size):
        pltpu.sync_copy(z128.at[pl.ds(0, size)], acc_sh.at[pl.ds(off, size)])
        pltpu.sync_copy(z16.at[pl.ds(0, size)], den_sh.at[pl.ds(off, size)])

    _striped_rows(s, _zero)
    pltpu.sync_copy(z16.at[pl.ds(0, CH)], sbufA)
    pltpu.sync_copy(z16.at[pl.ds(0, CH)], sbufB)
    plsc.subcore_barrier()

    per_sub = E // NSUB
    nchunks = per_sub // CH
    npairs = nchunks // 2
    base = s * per_sub
    iota = lax.iota(_i32, 16)
    zi = jnp.zeros((16,), _i32)
    oi = jnp.full((16,), 1, _i32)
    coff = c * N

    bufA = (edqA, srcA, dstA, src2A, arowA, browA, hrowA, sbufA, s2A,
            gsemA, isemA)
    bufB = (edqB, srcB, dstB, src2B, arowB, browB, hrowB, sbufB, s2B,
            gsemB, isemB)

    def issue(i, buf):
        edq, srcv, dstv, src2v, arow, brow, hrow, sbuf, s2v, gsem, isem = buf
        off = base + i * CH
        pltpu.make_async_copy(ed_h.at[pl.ds(off, CH)], edq, isem).wait()
        for j in range(CH // 16):
            r = 16 * j + iota
            sv_ = plsc.load_gather(edq, (r, zi))
            dv_ = plsc.load_gather(edq, (r, oi))
            srcv[pl.ds(16 * j, 16)] = sv_
            dstv[pl.ds(16 * j, 16)] = dv_
            src2v[pl.ds(16 * j, 16)] = sv_ + coff
        pltpu.async_copy(h2ab_h.at[src2v], hrow, gsem)
        pltpu.async_copy(a2_h.at[srcv], arow, gsem)
        pltpu.async_copy(a2_h.at[dstv], brow, gsem)

        @pl.when(i + 2 < nchunks)
        def _():
            pltpu.async_copy(ed_h.at[pl.ds(off + 2 * CH, CH)], edq, isem)

    def wait_g(buf):
        edq, srcv, dstv, src2v, arow, brow, hrow, sbuf, s2v, gsem, isem = buf
        pltpu.make_async_copy(h2ab_h.at[src2v], hrow, gsem).wait()
        pltpu.make_async_copy(a2_h.at[srcv], arow, gsem).wait()
        pltpu.make_async_copy(a2_h.at[dstv], brow, gsem).wait()

    def compute(buf):
        edq, srcv, dstv, src2v, arow, brow, hrow, sbuf, s2v, gsem, isem = buf
        for j in range(CH // 16):
            r = 16 * j + iota
            va = plsc.load_gather(arow, (r, zi))
            vb = plsc.load_gather(brow, (r, oi))
            xv = va + vb
            s16 = jnp.exp(jnp.maximum(xv, 0.2 * xv))
            plsc.store_scatter(sbuf, (r, zi), s16)
            s2v[pl.ds(16 * j, 16)] = s16

        def edge(j, cc):
            for u in range(2):
                jj = 2 * j + u
                sj = plsc.load_gather(s2v, (jnp.full((16,), jj, _i32),))
                for k in range(8):
                    hrow[jj, pl.ds(16 * k, 16)] = (
                        hrow[jj, pl.ds(16 * k, 16)] * sj)
            return cc

        lax.fori_loop(0, CH // 2, edge, 0)
        pltpu.sync_copy(sbuf, den_sh.at[dstv], add=True)
        pltpu.sync_copy(hrow, acc_sh.at[dstv], add=True)

    pltpu.async_copy(ed_h.at[pl.ds(base, CH)], edqA, isemA)
    pltpu.async_copy(ed_h.at[pl.ds(base + CH, CH)], edqB, isemB)
    issue(0, bufA)

    def pair(g, cc):
        i0 = 2 * g
        issue(i0 + 1, bufB)
        wait_g(bufA)
        compute(bufA)

        @pl.when(i0 + 2 < nchunks)
        def _():
            issue(i0 + 2, bufA)

        wait_g(bufB)
        compute(bufB)
        return cc

    lax.fori_loop(0, npairs, pair, 0)

    plsc.subcore_barrier()

    def _out0(off, size):
        pltpu.sync_copy(acc_sh.at[pl.ds(off, size)], houta.at[pl.ds(off, size)])
        pltpu.sync_copy(den_sh.at[pl.ds(off, size)], den2p.at[pl.ds(off, size)])

    def _out1(off, size):
        pltpu.sync_copy(acc_sh.at[pl.ds(off, size)], houtb.at[pl.ds(off, size)])

    @pl.when(c == 0)
    def _():
        _striped_rows(s, _out0)

    @pl.when(c == 1)
    def _():
        _striped_rows(s, _out1)

    half = nchunks // NCORE
    drow = browB
    alpha_v = s2A

    def achunk(a, cc):
        i = c * half + a
        off = base + i * CH
        pltpu.sync_copy(ed_h.at[pl.ds(off, CH)], edqA)
        for j in range(CH // 16):
            r = 16 * j + iota
            srcA[pl.ds(16 * j, 16)] = plsc.load_gather(edqA, (r, zi))
            dstA[pl.ds(16 * j, 16)] = plsc.load_gather(edqA, (r, oi))
        pltpu.async_copy(a2_h.at[srcA], arowA, gsemA)
        pltpu.async_copy(a2_h.at[dstA], browA, gsemA)
        pltpu.async_copy(den_sh.at[dstA], drow, dsem)
        pltpu.make_async_copy(a2_h.at[srcA], arowA, gsemA).wait()
        pltpu.make_async_copy(a2_h.at[dstA], browA, gsemA).wait()
        pltpu.make_async_copy(den_sh.at[dstA], drow, dsem).wait()
        for j in range(CH // 16):
            r = 16 * j + iota
            va = plsc.load_gather(arowA, (r, zi))
            vb = plsc.load_gather(browA, (r, oi))
            xv = va + vb
            s16 = jnp.exp(jnp.maximum(xv, 0.2 * xv))
            dv = plsc.load_gather(drow, (r, zi))
            alpha_v[pl.ds(16 * j, 16)] = s16 / (dv + 1e-16)
        pltpu.sync_copy(alpha_v, alpha_h.at[pl.ds(off, CH)])
        return cc

    lax.fori_loop(0, half, achunk, 0)


def _run_gat2(ed2, a2sd, h2ab, z128, z16):
    mesh = plsc.VectorSubcoreMesh(core_axis_name="c", subcore_axis_name="s",
                                  num_cores=NCORE, num_subcores=NSUB)
    buf = [
        pltpu.VMEM((CH, 2), _i32),
        pltpu.VMEM((CH,), _i32),
        pltpu.VMEM((CH,), _i32),
        pltpu.VMEM((CH,), _i32),
        pltpu.VMEM((CH, 16), _f32),
        pltpu.VMEM((CH, 16), _f32),
        pltpu.VMEM((CH, 128), _f32),
        pltpu.VMEM((CH, 16), _f32),
        pltpu.VMEM((CH,), _f32),
    ]
    fn = pl.kernel(
        _sc_gat2,
        out_type=[
            jax.ShapeDtypeStruct((N, 128), _f32),
            jax.ShapeDtypeStruct((N, 128), _f32),
            jax.ShapeDtypeStruct((N, 16), _f32),
            jax.ShapeDtypeStruct((E,), _f32),
        ],
        mesh=mesh,
        scratch_types=[
            pltpu.VMEM_SHARED((N, 128), _f32),
            pltpu.VMEM_SHARED((N, 16), _f32),
            *buf, *buf,
            pltpu.SemaphoreType.DMA,
            pltpu.SemaphoreType.DMA,
            pltpu.SemaphoreType.DMA,
            pltpu.SemaphoreType.DMA,
            pltpu.SemaphoreType.DMA,
        ],
        compiler_params=_SC_PARAMS,
    )
    return fn(ed2, a2sd, h2ab, z128, z16)


def _tc_head(ha_b, hb_b, den_b, bt_b, b2a, b2b, f1w, f1b, f2w, f2b,
             f3w, f3b, out_ref, pool_acc, cnt_acc):
    i = pl.program_id(0)

    @pl.when(i == 0)
    def _():
        pool_acc[...] = jnp.zeros_like(pool_acc)
        cnt_acc[...] = jnp.zeros_like(cnt_acc)

    den = den_b[...][:, :1]
    h2a = ha_b[...] / (den + 1e-16) + b2a[...]
    h2b = hb_b[...] / (den + 1e-16) + b2b[...]
    bt = bt_b[...].reshape(1, BN)
    g_iota = lax.broadcasted_iota(_i32, (G, BN), 0)
    oh = (g_iota == bt).astype(_f32)
    pool_acc[:, :128] += jnp.dot(oh, h2a, preferred_element_type=_f32)
    pool_acc[:, 128:] += jnp.dot(oh, h2b, preferred_element_type=_f32)
    cnt_acc[...] += jnp.dot(oh, jnp.ones((BN, 128), _f32),
                            preferred_element_type=_f32)

    @pl.when(i == (N // BN) - 1)
    def _():
        pooled = pool_acc[...] / jnp.maximum(cnt_acc[:, :1], 1.0)
        z = jnp.maximum(jnp.dot(pooled, f1w[...],
                                preferred_element_type=_f32) + f1b[...], 0.0)
        z = jnp.maximum(jnp.dot(z, f2w[...],
                                preferred_element_type=_f32) + f2b[...], 0.0)
        out_ref[...] = jnp.dot(z, f3w[...],
                               preferred_element_type=_f32) + f3b[...]


def _run_head(houta, houtb, den2p, batch3, b2a, b2b,
              fc1_w, fc1_b, fc2_w, fc2_b, fc3_w, fc3_b):
    return pl.pallas_call(
        _tc_head,
        grid=(N // BN,),
        in_specs=[
            pl.BlockSpec((BN, 128), lambda i: (i, 0)),
            pl.BlockSpec((BN, 128), lambda i: (i, 0)),
            pl.BlockSpec((BN, 16), lambda i: (i, 0)),
            pl.BlockSpec((1, 1, BN), lambda i: (i, 0, 0)),
            pl.BlockSpec((1, 128), lambda i: (0, 0)),
            pl.BlockSpec((1, 128), lambda i: (0, 0)),
            pl.BlockSpec((HID, 128), lambda i: (0, 0)),
            pl.BlockSpec((1, 128), lambda i: (0, 0)),
            pl.BlockSpec((128, 64), lambda i: (0, 0)),
            pl.BlockSpec((1, 64), lambda i: (0, 0)),
            pl.BlockSpec((64, NCLS), lambda i: (0, 0)),
            pl.BlockSpec((1, NCLS), lambda i: (0, 0)),
        ],
        out_specs=pl.BlockSpec((G, NCLS), lambda i: (0, 0)),
        out_shape=jax.ShapeDtypeStruct((G, NCLS), _f32),
        scratch_shapes=[
            pltpu.VMEM((G, HID), _f32),
            pltpu.VMEM((G, 128), _f32),
        ],
    )(houta, houtb, den2p, batch3, b2a, b2b,
      fc1_w, fc1_b, fc2_w, fc2_b, fc3_w, fc3_b)


def kernel(x, W1, a1_src, a1_dst, b1, W2, a2_src, a2_dst, b2,
           fc1_w, fc1_b, fc2_w, fc2_b, fc3_w, fc3_b, edge_index, batch):
    src = edge_index[0].astype(_i32)
    dst = edge_index[1].astype(_i32)

    hh = jnp.arange(64) // 8
    a1f_s = a1_src.reshape(64)
    a1f_d = a1_dst.reshape(64)
    eye8 = jnp.eye(8, dtype=_f32)
    ms_half = eye8[hh] * a1f_s[:, None]
    md_half = eye8[hh] * a1f_d[:, None]
    Ms = jnp.concatenate([ms_half, ms_half], axis=1)
    Md = jnp.concatenate([md_half, md_half], axis=1)
    E8 = jnp.repeat(eye8, 8, axis=1)
    A2m = jnp.zeros((HID, 16), _f32)
    A2m = A2m.at[:, 0].set(a2_src[0]).at[:, 1].set(a2_dst[0])

    z64 = jnp.zeros((RLAST, 64), _f32)
    z16 = jnp.zeros((RLAST, 16), _f32)
    z128 = jnp.zeros((RLAST, 128), _f32)

    ed2 = jnp.stack([src, dst], axis=1)
    hfeat, at, bt = _run_embed(x, W1, Ms, Md)
    acc0, acc1, den0, den1 = _run_gat1(ed2, at, bt, hfeat, z64, z16)
    h2ab3, a2sd = _run_mid(acc0, acc1, den0, den1,
                           b1.reshape(1, 64), E8, W2, A2m)
    h2ab = h2ab3.reshape(2 * N, 128)
    houta, houtb, den2p, alpha = _run_gat2(ed2, a2sd, h2ab, z128, z16)

    batch3 = batch.astype(_i32).reshape(N // BN, 1, BN)
    logits = _run_head(houta, houtb, den2p, batch3,
                       b2[:128].reshape(1, 128), b2[128:].reshape(1, 128),
                       fc1_w, fc1_b.reshape(1, 128),
                       fc2_w, fc2_b.reshape(1, 64),
                       fc3_w, fc3_b.reshape(1, NCLS))
    return logits, alpha.reshape(E, 1)


# speedup vs baseline: 1.5042x; 1.0809x over previous
import jax
import jax.numpy as jnp
from jax import lax
from jax.experimental import pallas as pl
from jax.experimental.pallas import tpu as pltpu
from jax.experimental.pallas import tpu_sc as plsc

N = 10000
E = 320000
D = 128
HID = 256
NCLS = 10
G = 64

NCORE = 2
NSUB = 16
CH = 80
BN = 1000
RSTRIPE = 624
RLAST = N - (NSUB - 1) * RSTRIPE

_f32 = jnp.float32
_i32 = jnp.int32

_SC_PARAMS = pltpu.CompilerParams(needs_layout_passes=False,
                                  use_tc_tiling_on_sc=False)


def _tc_embed(x_b, w1, ms, md, h_b, at_b, bt_b):
    h = jnp.dot(x_b[...], w1[...], preferred_element_type=_f32)
    h_b[...] = h
    at_b[...] = jnp.dot(h, ms[...], preferred_element_type=_f32)
    bt_b[...] = jnp.dot(h, md[...], preferred_element_type=_f32)


def _run_embed(x, W1, Ms, Md):
    return pl.pallas_call(
        _tc_embed,
        grid=(N // BN,),
        in_specs=[
            pl.BlockSpec((BN, D), lambda i: (i, 0)),
            pl.BlockSpec((D, 64), lambda i: (0, 0)),
            pl.BlockSpec((64, 16), lambda i: (0, 0)),
            pl.BlockSpec((64, 16), lambda i: (0, 0)),
        ],
        out_specs=[
            pl.BlockSpec((BN, 64), lambda i: (i, 0)),
            pl.BlockSpec((BN, 16), lambda i: (i, 0)),
            pl.BlockSpec((BN, 16), lambda i: (i, 0)),
        ],
        out_shape=[
            jax.ShapeDtypeStruct((N, 64), _f32),
            jax.ShapeDtypeStruct((N, 16), _f32),
            jax.ShapeDtypeStruct((N, 16), _f32),
        ],
    )(x, W1, Ms, Md)


def _striped_rows(s, do):
    @pl.when(s < NSUB - 1)
    def _():
        do(s * RSTRIPE, RSTRIPE)

    @pl.when(s == NSUB - 1)
    def _():
        do((NSUB - 1) * RSTRIPE, RLAST)


def _vcopy(src, dst, n):
    for j in range(n // 16):
        dst[pl.ds(16 * j, 16)] = src[pl.ds(16 * j, 16)]


def _sc_gat1(ed_h, at_h, bt_h, hf_h, z64, z16,
             acc0, acc1, den0, den1,
             acc_sh, den_sh,
             edqA, srcA, dstA, dstsA, arowA, browA, hrowA, svA, mA,
             edqB, srcB, dstB, dstsB, arowB, browB, hrowB, svB, mB,
             gsemA, gsemB, ssemA, ssemB, isemA, isemB):
    c = lax.axis_index("c")
    s = lax.axis_index("s")

    def _zero(off, size):
        pltpu.sync_copy(z64.at[pl.ds(0, size)], acc_sh.at[pl.ds(off, size)])
        pltpu.sync_copy(z16.at[pl.ds(0, size)], den_sh.at[pl.ds(off, size)])

    _striped_rows(s, _zero)
    plsc.subcore_barrier()

    wid = c * NSUB + s
    per_tile = E // (NCORE * NSUB)
    nchunks = per_tile // CH
    npairs = nchunks // 2
    base = wid * per_tile
    iota = lax.iota(_i32, 16)
    colsel = [2 * k + lax.shift_right_logical(iota, 3) for k in range(4)]
    zi = jnp.zeros((16,), _i32)
    oi = jnp.full((16,), 1, _i32)

    bufA = (edqA, srcA, dstA, dstsA, arowA, browA, hrowA, svA, mA,
            gsemA, ssemA, isemA)
    bufB = (edqB, srcB, dstB, dstsB, arowB, browB, hrowB, svB, mB,
            gsemB, ssemB, isemB)

    def issue(i, buf):
        edq, srcv, dstv, dsts, arow, brow, hrow, sv, m, gsem, ssem, isem = buf
        off = base + i * CH
        pltpu.make_async_copy(ed_h.at[pl.ds(off, CH)], edq, isem).wait()
        for j in range(CH // 16):
            r = 16 * j + iota
            sv_ = plsc.load_gather(edq, (r, zi))
            dv_ = plsc.load_gather(edq, (r, oi))
            srcv[pl.ds(16 * j, 16)] = sv_
            dstv[pl.ds(16 * j, 16)] = dv_
        pltpu.async_copy(hf_h.at[srcv], hrow, gsem)
        pltpu.async_copy(at_h.at[srcv], arow, gsem)
        pltpu.async_copy(bt_h.at[dstv], brow, gsem)

        @pl.when(i + 2 < nchunks)
        def _():
            pltpu.async_copy(ed_h.at[pl.ds(off + 2 * CH, CH)], edq, isem)

    def wait_g(buf):
        edq, srcv, dstv, dsts, arow, brow, hrow, sv, m, gsem, ssem, isem = buf
        pltpu.make_async_copy(hf_h.at[srcv], hrow, gsem).wait()
        pltpu.make_async_copy(at_h.at[srcv], arow, gsem).wait()
        pltpu.make_async_copy(bt_h.at[dstv], brow, gsem).wait()

    def compute(buf):
        edq, srcv, dstv, dsts, arow, brow, hrow, sv, m, gsem, ssem, isem = buf
        _vcopy(dstv, dsts, CH)

        def edge_s(j, cc):
            for u in range(2):
                jj = 2 * j + u
                xv = arow[jj, :] + brow[jj, :]
                sv[jj, :] = jnp.exp(jnp.maximum(xv, 0.2 * xv))
            return cc

        lax.fori_loop(0, CH // 2, edge_s, 0)

        def edge_m(j, cc):
            for u in range(2):
                jj = 2 * j + u
                jv = jnp.full((16,), jj, _i32)
                for k in range(4):
                    svv = plsc.load_gather(sv, (jv, colsel[k]))
                    m[jj, pl.ds(16 * k, 16)] = (
                        hrow[jj, pl.ds(16 * k, 16)] * svv)
            return cc

        lax.fori_loop(0, CH // 2, edge_m, 0)

    def issue_s(buf):
        edq, srcv, dstv, dsts, arow, brow, hrow, sv, m, gsem, ssem, isem = buf
        pltpu.async_copy(sv, den_sh.at[dsts], ssem, add=True)
        pltpu.async_copy(m, acc_sh.at[dsts], ssem, add=True)

    def wait_s(buf):
        edq, srcv, dstv, dsts, arow, brow, hrow, sv, m, gsem, ssem, isem = buf
        pltpu.make_async_copy(sv, den_sh.at[dsts], ssem).wait()
        pltpu.make_async_copy(m, acc_sh.at[dsts], ssem).wait()

    pltpu.async_copy(ed_h.at[pl.ds(base, CH)], edqA, isemA)
    pltpu.async_copy(ed_h.at[pl.ds(base + CH, CH)], edqB, isemB)
    issue(0, bufA)
    issue(1, bufB)

    def pair(g, cc):
        i0 = 2 * g
        wait_g(bufA)

        @pl.when(g > 0)
        def _():
            wait_s(bufA)

        compute(bufA)
        issue_s(bufA)

        @pl.when(i0 + 2 < nchunks)
        def _():
            issue(i0 + 2, bufA)

        wait_g(bufB)

        @pl.when(g > 0)
        def _():
            wait_s(bufB)

        compute(bufB)
        issue_s(bufB)

        @pl.when(i0 + 3 < nchunks)
        def _():
            issue(i0 + 3, bufB)

        return cc

    lax.fori_loop(0, npairs, pair, 0)
    if nchunks % 2 == 1:
        wait_g(bufA)
        wait_s(bufA)
        compute(bufA)
        issue_s(bufA)
    wait_s(bufA)
    wait_s(bufB)

    plsc.subcore_barrier()

    def _out0(off, size):
        pltpu.sync_copy(acc_sh.at[pl.ds(off, size)], acc0.at[pl.ds(off, size)])
        pltpu.sync_copy(den_sh.at[pl.ds(off, size)], den0.at[pl.ds(off, size)])

    def _out1(off, size):
        pltpu.sync_copy(acc_sh.at[pl.ds(off, size)], acc1.at[pl.ds(off, size)])
        pltpu.sync_copy(den_sh.at[pl.ds(off, size)], den1.at[pl.ds(off, size)])

    @pl.when(c == 0)
    def _():
        _striped_rows(s, _out0)

    @pl.when(c == 1)
    def _():
        _striped_rows(s, _out1)


def _run_gat1(ed2, at, bt, hfeat, z64, z16):
    mesh = plsc.VectorSubcoreMesh(core_axis_name="c", subcore_axis_name="s",
                                  num_cores=NCORE, num_subcores=NSUB)
    buf = [
        pltpu.VMEM((CH, 2), _i32),
        pltpu.VMEM((CH,), _i32),
        pltpu.VMEM((CH,), _i32),
        pltpu.VMEM((CH,), _i32),
        pltpu.VMEM((CH, 16), _f32),
        pltpu.VMEM((CH, 16), _f32),
        pltpu.VMEM((CH, 64), _f32),
        pltpu.VMEM((CH, 16), _f32),
        pltpu.VMEM((CH, 64), _f32),
    ]
    fn = pl.kernel(
        _sc_gat1,
        out_type=[
            jax.ShapeDtypeStruct((N, 64), _f32),
            jax.ShapeDtypeStruct((N, 64), _f32),
            jax.ShapeDtypeStruct((N, 16), _f32),
            jax.ShapeDtypeStruct((N, 16), _f32),
        ],
        mesh=mesh,
        scratch_types=[
            pltpu.VMEM_SHARED((N, 64), _f32),
            pltpu.VMEM_SHARED((N, 16), _f32),
            *buf, *buf,
            pltpu.SemaphoreType.DMA,
            pltpu.SemaphoreType.DMA,
            pltpu.SemaphoreType.DMA,
            pltpu.SemaphoreType.DMA,
            pltpu.SemaphoreType.DMA,
            pltpu.SemaphoreType.DMA,
        ],
        compiler_params=_SC_PARAMS,
    )
    return fn(ed2, at, bt, hfeat, z64, z16)


def _tc_mid(a0_b, a1_b, d0_b, d1_b, b1_r, e8, w2, a2m, h2ab_b, a2sd_b):
    acc = a0_b[...] + a1_b[...]
    den8 = (d0_b[...] + d1_b[...])[:, :8]
    denx = jnp.dot(den8, e8[...], preferred_element_type=_f32)
    h1 = jnp.maximum(acc / (denx + 1e-16) + b1_r[...], 0.0)
    h2pre = jnp.dot(h1, w2[...], preferred_element_type=_f32)
    h2ab_b[0, :, :] = h2pre[:, :128]
    h2ab_b[1, :, :] = h2pre[:, 128:]
    a2sd_b[...] = jnp.dot(h2pre, a2m[...], preferred_element_type=_f32)


def _run_mid(acc0, acc1, den0, den1, b1r, E8, W2, A2m):
    return pl.pallas_call(
        _tc_mid,
        grid=(N // BN,),
        in_specs=[
            pl.BlockSpec((BN, 64), lambda i: (i, 0)),
            pl.BlockSpec((BN, 64), lambda i: (i, 0)),
            pl.BlockSpec((BN, 16), lambda i: (i, 0)),
            pl.BlockSpec((BN, 16), lambda i: (i, 0)),
            pl.BlockSpec((1, 64), lambda i: (0, 0)),
            pl.BlockSpec((8, 64), lambda i: (0, 0)),
            pl.BlockSpec((64, HID), lambda i: (0, 0)),
            pl.BlockSpec((HID, 16), lambda i: (0, 0)),
        ],
        out_specs=[
            pl.BlockSpec((2, BN, 128), lambda i: (0, i, 0)),
            pl.BlockSpec((BN, 16), lambda i: (i, 0)),
        ],
        out_shape=[
            jax.ShapeDtypeStruct((2, N, 128), _f32),
            jax.ShapeDtypeStruct((N, 16), _f32),
        ],
    )(acc0, acc1, den0, den1, b1r, E8, W2, A2m)


def _sc_gat2(ed_h, a2_h, h2ab_h, z128, z16,
             houta, houtb, den2p, alpha_h,
             acc_sh, den_sh,
             edqA, srcA, dstA, src2A, arowA, browA, hrowA, sbufA, s2A,
             edqB, srcB, dstB, src2B, arowB, browB, hrowB, sbufB, s2B,
             gsemA, gsemB, dsem, isemA, isemB):
    c = lax.axis_index("c")
    s = lax.axis_index("s")

    def _zero(off, size):
        pltpu.sync_copy(z128.at[pl.ds(0, size)], acc_sh.at[pl.ds(off, size)])
        pltpu.sync_copy(z16.at[pl.ds(0, size)], den_sh.at[pl.ds(off, size)])

    _striped_rows(s, _zero)
    pltpu.sync_copy(z16.at[pl.ds(0, CH)], sbufA)
    pltpu.sync_copy(z16.at[pl.ds(0, CH)], sbufB)
    plsc.subcore_barrier()

    per_sub = E // NSUB
    nchunks = per_sub // CH
    npairs = nchunks // 2
    base = s * per_sub
    iota = lax.iota(_i32, 16)
    zi = jnp.zeros((16,), _i32)
    oi = jnp.full((16,), 1, _i32)
    coff = c * N

    bufA = (edqA, srcA, dstA, src2A, arowA, browA, hrowA, sbufA, s2A,
            gsemA, isemA)
    bufB = (edqB, srcB, dstB, src2B, arowB, browB, hrowB, sbufB, s2B,
            gsemB, isemB)

    def issue(i, buf):
        edq, srcv, dstv, src2v, arow, brow, hrow, sbuf, s2v, gsem, isem = buf
        off = base + i * CH
        pltpu.make_async_copy(ed_h.at[pl.ds(off, CH)], edq, isem).wait()
        for j in range(CH // 16):
            r = 16 * j + iota
            sv_ = plsc.load_gather(edq, (r, zi))
            dv_ = plsc.load_gather(edq, (r, oi))
            srcv[pl.ds(16 * j, 16)] = sv_
            dstv[pl.ds(16 * j, 16)] = dv_
            src2v[pl.ds(16 * j, 16)] = sv_ + coff
        pltpu.async_copy(h2ab_h.at[src2v], hrow, gsem)
        pltpu.async_copy(a2_h.at[srcv], arow, gsem)
        pltpu.async_copy(a2_h.at[dstv], brow, gsem)

        @pl.when(i + 2 < nchunks)
        def _():
            pltpu.async_copy(ed_h.at[pl.ds(off + 2 * CH, CH)], edq, isem)

    def wait_g(buf):
        edq, srcv, dstv, src2v, arow, brow, hrow, sbuf, s2v, gsem, isem = buf
        pltpu.make_async_copy(h2ab_h.at[src2v], hrow, gsem).wait()
        pltpu.make_async_copy(a2_h.at[srcv], arow, gsem).wait()
        pltpu.make_async_copy(a2_h.at[dstv], brow, gsem).wait()

    def compute(buf):
        edq, srcv, dstv, src2v, arow, brow, hrow, sbuf, s2v, gsem, isem = buf
        for j in range(CH // 16):
            r = 16 * j + iota
            va = plsc.load_gather(arow, (r, zi))
            vb = plsc.load_gather(brow, (r, oi))
            xv = va + vb
            s16 = jnp.exp(jnp.maximum(xv, 0.2 * xv))
            plsc.store_scatter(sbuf, (r, zi), s16)
            s2v[pl.ds(16 * j, 16)] = s16

        def edge(j, cc):
            for u in range(2):
                jj = 2 * j + u
                sj = plsc.load_gather(s2v, (jnp.full((16,), jj, _i32),))
                for k in range(8):
                    hrow[jj, pl.ds(16 * k, 16)] = (
                        hrow[jj, pl.ds(16 * k, 16)] * sj)
            return cc

        lax.fori_loop(0, CH // 2, edge, 0)
        pltpu.sync_copy(sbuf, den_sh.at[dstv], add=True)
        pltpu.sync_copy(hrow, acc_sh.at[dstv], add=True)

    pltpu.async_copy(ed_h.at[pl.ds(base, CH)], edqA, isemA)
    pltpu.async_copy(ed_h.at[pl.ds(base + CH, CH)], edqB, isemB)
    issue(0, bufA)

    def pair(g, cc):
        i0 = 2 * g
        issue(i0 + 1, bufB)
        wait_g(bufA)
        compute(bufA)

        @pl.when(i0 + 2 < nchunks)
        def _():
            issue(i0 + 2, bufA)

        wait_g(bufB)
        compute(bufB)
        return cc

    lax.fori_loop(0, npairs, pair, 0)

    plsc.subcore_barrier()

    def _out0(off, size):
        pltpu.sync_copy(acc_sh.at[pl.ds(off, size)], houta.at[pl.ds(off, size)])
        pltpu.sync_copy(den_sh.at[pl.ds(off, size)], den2p.at[pl.ds(off, size)])

    def _out1(off, size):
        pltpu.sync_copy(acc_sh.at[pl.ds(off, size)], houtb.at[pl.ds(off, size)])

    @pl.when(c == 0)
    def _():
        _striped_rows(s, _out0)

    @pl.when(c == 1)
    def _():
        _striped_rows(s, _out1)

    half = nchunks // NCORE
    ahalf_pairs = half // 2
    setA = (edqA, srcA, dstA, arowA, browA, sbufA, s2A, gsemA, isemA)
    setB = (edqB, srcB, dstB, arowB, browB, sbufB, s2B, gsemB, isemB)

    def aissue(a, st):
        edq, srcv, dstv, arow, brow, drow, av, gsem, dsm = st
        off = base + (c * half + a) * CH
        pltpu.sync_copy(ed_h.at[pl.ds(off, CH)], edq)
        for j in range(CH // 16):
            r = 16 * j + iota
            srcv[pl.ds(16 * j, 16)] = plsc.load_gather(edq, (r, zi))
            dstv[pl.ds(16 * j, 16)] = plsc.load_gather(edq, (r, oi))
        pltpu.async_copy(a2_h.at[srcv], arow, gsem)
        pltpu.async_copy(a2_h.at[dstv], brow, gsem)
        pltpu.async_copy(den_sh.at[dstv], drow, dsm)

    def acompute(a, st):
        edq, srcv, dstv, arow, brow, drow, av, gsem, dsm = st
        pltpu.make_async_copy(a2_h.at[srcv], arow, gsem).wait()
        pltpu.make_async_copy(a2_h.at[dstv], brow, gsem).wait()
        pltpu.make_async_copy(den_sh.at[dstv], drow, dsm).wait()
        for j in range(CH // 16):
            r = 16 * j + iota
            va = plsc.load_gather(arow, (r, zi))
            vb = plsc.load_gather(brow, (r, oi))
            xv = va + vb
            s16 = jnp.exp(jnp.maximum(xv, 0.2 * xv))
            dv = plsc.load_gather(drow, (r, zi))
            av[pl.ds(16 * j, 16)] = s16 / (dv + 1e-16)
        off = base + (c * half + a) * CH
        pltpu.sync_copy(av, alpha_h.at[pl.ds(off, CH)])

    aissue(0, setA)

    def apair(g, cc):
        a0 = 2 * g
        aissue(a0 + 1, setB)
        acompute(a0, setA)

        @pl.when(a0 + 2 < half)
        def _():
            aissue(a0 + 2, setA)

        acompute(a0 + 1, setB)
        return cc

    lax.fori_loop(0, ahalf_pairs, apair, 0)
    if half % 2 == 1:
        acompute(half - 1, setA)


def _run_gat2(ed2, a2sd, h2ab, z128, z16):
    mesh = plsc.VectorSubcoreMesh(core_axis_name="c", subcore_axis_name="s",
                                  num_cores=NCORE, num_subcores=NSUB)
    buf = [
        pltpu.VMEM((CH, 2), _i32),
        pltpu.VMEM((CH,), _i32),
        pltpu.VMEM((CH,), _i32),
        pltpu.VMEM((CH,), _i32),
        pltpu.VMEM((CH, 16), _f32),
        pltpu.VMEM((CH, 16), _f32),
        pltpu.VMEM((CH, 128), _f32),
        pltpu.VMEM((CH, 16), _f32),
        pltpu.VMEM((CH,), _f32),
    ]
    fn = pl.kernel(
        _sc_gat2,
        out_type=[
            jax.ShapeDtypeStruct((N, 128), _f32),
            jax.ShapeDtypeStruct((N, 128), _f32),
            jax.ShapeDtypeStruct((N, 16), _f32),
            jax.ShapeDtypeStruct((E,), _f32),
        ],
        mesh=mesh,
        scratch_types=[
            pltpu.VMEM_SHARED((N, 128), _f32),
            pltpu.VMEM_SHARED((N, 16), _f32),
            *buf, *buf,
            pltpu.SemaphoreType.DMA,
            pltpu.SemaphoreType.DMA,
            pltpu.SemaphoreType.DMA,
            pltpu.SemaphoreType.DMA,
            pltpu.SemaphoreType.DMA,
        ],
        compiler_params=_SC_PARAMS,
    )
    return fn(ed2, a2sd, h2ab, z128, z16)


def _tc_head(ha_b, hb_b, den_b, bt_b, b2a, b2b, f1w, f1b, f2w, f2b,
             f3w, f3b, out_ref, pool_acc, cnt_acc):
    i = pl.program_id(0)

    @pl.when(i == 0)
    def _():
        pool_acc[...] = jnp.zeros_like(pool_acc)
        cnt_acc[...] = jnp.zeros_like(cnt_acc)

    den = den_b[...][:, :1]
    h2a = ha_b[...] / (den + 1e-16) + b2a[...]
    h2b = hb_b[...] / (den + 1e-16) + b2b[...]
    bt = bt_b[...].reshape(1, BN)
    g_iota = lax.broadcasted_iota(_i32, (G, BN), 0)
    oh = (g_iota == bt).astype(_f32)
    pool_acc[:, :128] += jnp.dot(oh, h2a, preferred_element_type=_f32)
    pool_acc[:, 128:] += jnp.dot(oh, h2b, preferred_element_type=_f32)
    cnt_acc[...] += jnp.dot(oh, jnp.ones((BN, 128), _f32),
                            preferred_element_type=_f32)

    @pl.when(i == (N // BN) - 1)
    def _():
        pooled = pool_acc[...] / jnp.maximum(cnt_acc[:, :1], 1.0)
        z = jnp.maximum(jnp.dot(pooled, f1w[...],
                                preferred_element_type=_f32) + f1b[...], 0.0)
        z = jnp.maximum(jnp.dot(z, f2w[...],
                                preferred_element_type=_f32) + f2b[...], 0.0)
        out_ref[...] = jnp.dot(z, f3w[...],
                               preferred_element_type=_f32) + f3b[...]


def _run_head(houta, houtb, den2p, batch3, b2a, b2b,
              fc1_w, fc1_b, fc2_w, fc2_b, fc3_w, fc3_b):
    return pl.pallas_call(
        _tc_head,
        grid=(N // BN,),
        in_specs=[
            pl.BlockSpec((BN, 128), lambda i: (i, 0)),
            pl.BlockSpec((BN, 128), lambda i: (i, 0)),
            pl.BlockSpec((BN, 16), lambda i: (i, 0)),
            pl.BlockSpec((1, 1, BN), lambda i: (i, 0, 0)),
            pl.BlockSpec((1, 128), lambda i: (0, 0)),
            pl.BlockSpec((1, 128), lambda i: (0, 0)),
            pl.BlockSpec((HID, 128), lambda i: (0, 0)),
            pl.BlockSpec((1, 128), lambda i: (0, 0)),
            pl.BlockSpec((128, 64), lambda i: (0, 0)),
            pl.BlockSpec((1, 64), lambda i: (0, 0)),
            pl.BlockSpec((64, NCLS), lambda i: (0, 0)),
            pl.BlockSpec((1, NCLS), lambda i: (0, 0)),
        ],
        out_specs=pl.BlockSpec((G, NCLS), lambda i: (0, 0)),
        out_shape=jax.ShapeDtypeStruct((G, NCLS), _f32),
        scratch_shapes=[
            pltpu.VMEM((G, HID), _f32),
            pltpu.VMEM((G, 128), _f32),
        ],
    )(houta, houtb, den2p, batch3, b2a, b2b,
      fc1_w, fc1_b, fc2_w, fc2_b, fc3_w, fc3_b)


def kernel(x, W1, a1_src, a1_dst, b1, W2, a2_src, a2_dst, b2,
           fc1_w, fc1_b, fc2_w, fc2_b, fc3_w, fc3_b, edge_index, batch):
    src = edge_index[0].astype(_i32)
    dst = edge_index[1].astype(_i32)

    hh = jnp.arange(64) // 8
    a1f_s = a1_src.reshape(64)
    a1f_d = a1_dst.reshape(64)
    eye8 = jnp.eye(8, dtype=_f32)
    ms_half = eye8[hh] * a1f_s[:, None]
    md_half = eye8[hh] * a1f_d[:, None]
    Ms = jnp.concatenate([ms_half, ms_half], axis=1)
    Md = jnp.concatenate([md_half, md_half], axis=1)
    E8 = jnp.repeat(eye8, 8, axis=1)
    A2m = jnp.zeros((HID, 16), _f32)
    A2m = A2m.at[:, 0].set(a2_src[0]).at[:, 1].set(a2_dst[0])

    z64 = jnp.zeros((RLAST, 64), _f32)
    z16 = jnp.zeros((RLAST, 16), _f32)
    z128 = jnp.zeros((RLAST, 128), _f32)

    ed2 = jnp.stack([src, dst], axis=1)
    hfeat, at, bt = _run_embed(x, W1, Ms, Md)
    acc0, acc1, den0, den1 = _run_gat1(ed2, at, bt, hfeat, z64, z16)
    h2ab3, a2sd = _run_mid(acc0, acc1, den0, den1,
                           b1.reshape(1, 64), E8, W2, A2m)
    h2ab = h2ab3.reshape(2 * N, 128)
    houta, houtb, den2p, alpha = _run_gat2(ed2, a2sd, h2ab, z128, z16)

    batch3 = batch.astype(_i32).reshape(N // BN, 1, BN)
    logits = _run_head(houta, houtb, den2p, batch3,
                       b2[:128].reshape(1, 128), b2[128:].reshape(1, 128),
                       fc1_w, fc1_b.reshape(1, 128),
                       fc2_w, fc2_b.reshape(1, 64),
                       fc3_w, fc3_b.reshape(1, NCLS))
    return logits, alpha.reshape(E, 1)


# speedup vs baseline: 1.5248x; 1.0137x over previous
import jax
import jax.numpy as jnp
from jax import lax
from jax.experimental import pallas as pl
from jax.experimental.pallas import tpu as pltpu
from jax.experimental.pallas import tpu_sc as plsc

N = 10000
E = 320000
D = 128
HID = 256
NCLS = 10
G = 64

NCORE = 2
NSUB = 16
CH = 80
BN = 1000
RSTRIPE = 624
RLAST = N - (NSUB - 1) * RSTRIPE

_f32 = jnp.float32
_i32 = jnp.int32

_SC_PARAMS = pltpu.CompilerParams(needs_layout_passes=False,
                                  use_tc_tiling_on_sc=False)


def _tc_embed(x_b, w1, ms, md, h_b, at_b, bt_b):
    h = jnp.dot(x_b[...], w1[...], preferred_element_type=_f32)
    h_b[...] = h
    at_b[...] = jnp.dot(h, ms[...], preferred_element_type=_f32)
    bt_b[...] = jnp.dot(h, md[...], preferred_element_type=_f32)


def _run_embed(x, W1, Ms, Md):
    return pl.pallas_call(
        _tc_embed,
        grid=(N // BN,),
        in_specs=[
            pl.BlockSpec((BN, D), lambda i: (i, 0)),
            pl.BlockSpec((D, 64), lambda i: (0, 0)),
            pl.BlockSpec((64, 16), lambda i: (0, 0)),
            pl.BlockSpec((64, 16), lambda i: (0, 0)),
        ],
        out_specs=[
            pl.BlockSpec((BN, 64), lambda i: (i, 0)),
            pl.BlockSpec((BN, 16), lambda i: (i, 0)),
            pl.BlockSpec((BN, 16), lambda i: (i, 0)),
        ],
        out_shape=[
            jax.ShapeDtypeStruct((N, 64), _f32),
            jax.ShapeDtypeStruct((N, 16), _f32),
            jax.ShapeDtypeStruct((N, 16), _f32),
        ],
    )(x, W1, Ms, Md)


def _striped_rows(s, do):
    @pl.when(s < NSUB - 1)
    def _():
        do(s * RSTRIPE, RSTRIPE)

    @pl.when(s == NSUB - 1)
    def _():
        do((NSUB - 1) * RSTRIPE, RLAST)


def _vcopy(src, dst, n):
    for j in range(n // 16):
        dst[pl.ds(16 * j, 16)] = src[pl.ds(16 * j, 16)]


def _sc_gat1(ed_h, at_h, bt_h, hf_h, z64, z16,
             acc0, acc1, den0, den1,
             acc_sh, den_sh,
             edqA, srcA, dstA, dstsA, arowA, browA, hrowA, svA, mA,
             edqB, srcB, dstB, dstsB, arowB, browB, hrowB, svB, mB,
             gsemA, gsemB, ssemA, ssemB, isemA, isemB):
    c = lax.axis_index("c")
    s = lax.axis_index("s")

    def _zero(off, size):
        pltpu.sync_copy(z64.at[pl.ds(0, size)], acc_sh.at[pl.ds(off, size)])
        pltpu.sync_copy(z16.at[pl.ds(0, size)], den_sh.at[pl.ds(off, size)])

    _striped_rows(s, _zero)
    plsc.subcore_barrier()

    wid = c * NSUB + s
    per_tile = E // (NCORE * NSUB)
    nchunks = per_tile // CH
    npairs = nchunks // 2
    base = wid * per_tile
    iota = lax.iota(_i32, 16)
    colsel = [2 * k + lax.shift_right_logical(iota, 3) for k in range(4)]
    zi = jnp.zeros((16,), _i32)
    oi = jnp.full((16,), 1, _i32)

    bufA = (edqA, srcA, dstA, dstsA, arowA, browA, hrowA, svA, mA,
            gsemA, ssemA, isemA)
    bufB = (edqB, srcB, dstB, dstsB, arowB, browB, hrowB, svB, mB,
            gsemB, ssemB, isemB)

    def issue(i, buf):
        edq, srcv, dstv, dsts, arow, brow, hrow, sv, m, gsem, ssem, isem = buf
        off = base + i * CH
        pltpu.make_async_copy(ed_h.at[pl.ds(off, CH)], edq, isem).wait()
        for j in range(CH // 16):
            r = 16 * j + iota
            sv_ = plsc.load_gather(edq, (r, zi))
            dv_ = plsc.load_gather(edq, (r, oi))
            srcv[pl.ds(16 * j, 16)] = sv_
            dstv[pl.ds(16 * j, 16)] = dv_
        pltpu.async_copy(hf_h.at[srcv], hrow, gsem)
        pltpu.async_copy(at_h.at[srcv], arow, gsem)
        pltpu.async_copy(bt_h.at[dstv], brow, gsem)

        @pl.when(i + 2 < nchunks)
        def _():
            pltpu.async_copy(ed_h.at[pl.ds(off + 2 * CH, CH)], edq, isem)

    def wait_g(buf):
        edq, srcv, dstv, dsts, arow, brow, hrow, sv, m, gsem, ssem, isem = buf
        pltpu.make_async_copy(hf_h.at[srcv], hrow, gsem).wait()
        pltpu.make_async_copy(at_h.at[srcv], arow, gsem).wait()
        pltpu.make_async_copy(bt_h.at[dstv], brow, gsem).wait()

    def compute(buf):
        edq, srcv, dstv, dsts, arow, brow, hrow, sv, m, gsem, ssem, isem = buf
        _vcopy(dstv, dsts, CH)

        def edge_s(j, cc):
            for u in range(4):
                jj = 4 * j + u
                xv = arow[jj, :] + brow[jj, :]
                sv[jj, :] = jnp.exp(jnp.maximum(xv, 0.2 * xv))
            return cc

        lax.fori_loop(0, CH // 4, edge_s, 0)

        def edge_m(j, cc):
            for u in range(4):
                jj = 4 * j + u
                jv = jnp.full((16,), jj, _i32)
                for k in range(4):
                    svv = plsc.load_gather(sv, (jv, colsel[k]))
                    m[jj, pl.ds(16 * k, 16)] = (
                        hrow[jj, pl.ds(16 * k, 16)] * svv)
            return cc

        lax.fori_loop(0, CH // 4, edge_m, 0)

    def issue_s(buf):
        edq, srcv, dstv, dsts, arow, brow, hrow, sv, m, gsem, ssem, isem = buf
        pltpu.async_copy(sv, den_sh.at[dsts], ssem, add=True)
        pltpu.async_copy(m, acc_sh.at[dsts], ssem, add=True)

    def wait_s(buf):
        edq, srcv, dstv, dsts, arow, brow, hrow, sv, m, gsem, ssem, isem = buf
        pltpu.make_async_copy(sv, den_sh.at[dsts], ssem).wait()
        pltpu.make_async_copy(m, acc_sh.at[dsts], ssem).wait()

    pltpu.async_copy(ed_h.at[pl.ds(base, CH)], edqA, isemA)
    pltpu.async_copy(ed_h.at[pl.ds(base + CH, CH)], edqB, isemB)
    issue(0, bufA)
    issue(1, bufB)

    def pair(g, cc):
        i0 = 2 * g
        wait_g(bufA)

        @pl.when(g > 0)
        def _():
            wait_s(bufA)

        compute(bufA)
        issue_s(bufA)

        @pl.when(i0 + 2 < nchunks)
        def _():
            issue(i0 + 2, bufA)

        wait_g(bufB)

        @pl.when(g > 0)
        def _():
            wait_s(bufB)

        compute(bufB)
        issue_s(bufB)

        @pl.when(i0 + 3 < nchunks)
        def _():
            issue(i0 + 3, bufB)

        return cc

    lax.fori_loop(0, npairs, pair, 0)
    if nchunks % 2 == 1:
        wait_g(bufA)
        wait_s(bufA)
        compute(bufA)
        issue_s(bufA)
    wait_s(bufA)
    wait_s(bufB)

    plsc.subcore_barrier()

    def _out0(off, size):
        pltpu.sync_copy(acc_sh.at[pl.ds(off, size)], acc0.at[pl.ds(off, size)])
        pltpu.sync_copy(den_sh.at[pl.ds(off, size)], den0.at[pl.ds(off, size)])

    def _out1(off, size):
        pltpu.sync_copy(acc_sh.at[pl.ds(off, size)], acc1.at[pl.ds(off, size)])
        pltpu.sync_copy(den_sh.at[pl.ds(off, size)], den1.at[pl.ds(off, size)])

    @pl.when(c == 0)
    def _():
        _striped_rows(s, _out0)

    @pl.when(c == 1)
    def _():
        _striped_rows(s, _out1)


def _run_gat1(ed2, at, bt, hfeat, z64, z16):
    mesh = plsc.VectorSubcoreMesh(core_axis_name="c", subcore_axis_name="s",
                                  num_cores=NCORE, num_subcores=NSUB)
    buf = [
        pltpu.VMEM((CH, 2), _i32),
        pltpu.VMEM((CH,), _i32),
        pltpu.VMEM((CH,), _i32),
        pltpu.VMEM((CH,), _i32),
        pltpu.VMEM((CH, 16), _f32),
        pltpu.VMEM((CH, 16), _f32),
        pltpu.VMEM((CH, 64), _f32),
        pltpu.VMEM((CH, 16), _f32),
        pltpu.VMEM((CH, 64), _f32),
    ]
    fn = pl.kernel(
        _sc_gat1,
        out_type=[
            jax.ShapeDtypeStruct((N, 64), _f32),
            jax.ShapeDtypeStruct((N, 64), _f32),
            jax.ShapeDtypeStruct((N, 16), _f32),
            jax.ShapeDtypeStruct((N, 16), _f32),
        ],
        mesh=mesh,
        scratch_types=[
            pltpu.VMEM_SHARED((N, 64), _f32),
            pltpu.VMEM_SHARED((N, 16), _f32),
            *buf, *buf,
            pltpu.SemaphoreType.DMA,
            pltpu.SemaphoreType.DMA,
            pltpu.SemaphoreType.DMA,
            pltpu.SemaphoreType.DMA,
            pltpu.SemaphoreType.DMA,
            pltpu.SemaphoreType.DMA,
        ],
        compiler_params=_SC_PARAMS,
    )
    return fn(ed2, at, bt, hfeat, z64, z16)


def _tc_mid(a0_b, a1_b, d0_b, d1_b, b1_r, e8, w2, a2m, h2ab_b, a2sd_b):
    acc = a0_b[...] + a1_b[...]
    den8 = (d0_b[...] + d1_b[...])[:, :8]
    denx = jnp.dot(den8, e8[...], preferred_element_type=_f32)
    h1 = jnp.maximum(acc / (denx + 1e-16) + b1_r[...], 0.0)
    h2pre = jnp.dot(h1, w2[...], preferred_element_type=_f32)
    h2ab_b[0, :, :] = h2pre[:, :128]
    h2ab_b[1, :, :] = h2pre[:, 128:]
    a2sd_b[...] = jnp.dot(h2pre, a2m[...], preferred_element_type=_f32)


def _run_mid(acc0, acc1, den0, den1, b1r, E8, W2, A2m):
    return pl.pallas_call(
        _tc_mid,
        grid=(N // BN,),
        in_specs=[
            pl.BlockSpec((BN, 64), lambda i: (i, 0)),
            pl.BlockSpec((BN, 64), lambda i: (i, 0)),
            pl.BlockSpec((BN, 16), lambda i: (i, 0)),
            pl.BlockSpec((BN, 16), lambda i: (i, 0)),
            pl.BlockSpec((1, 64), lambda i: (0, 0)),
            pl.BlockSpec((8, 64), lambda i: (0, 0)),
            pl.BlockSpec((64, HID), lambda i: (0, 0)),
            pl.BlockSpec((HID, 16), lambda i: (0, 0)),
        ],
        out_specs=[
            pl.BlockSpec((2, BN, 128), lambda i: (0, i, 0)),
            pl.BlockSpec((BN, 16), lambda i: (i, 0)),
        ],
        out_shape=[
            jax.ShapeDtypeStruct((2, N, 128), _f32),
            jax.ShapeDtypeStruct((N, 16), _f32),
        ],
    )(acc0, acc1, den0, den1, b1r, E8, W2, A2m)


def _sc_gat2(ed_h, a2_h, h2ab_h, z128, z16,
             houta, houtb, den2p, alpha_h,
             acc_sh, den_sh,
             edqA, srcA, dstA, src2A, arowA, browA, hrowA, sbufA, s2A,
             edqB, srcB, dstB, src2B, arowB, browB, hrowB, sbufB, s2B,
             gsemA, gsemB, dsem, isemA, isemB):
    c = lax.axis_index("c")
    s = lax.axis_index("s")

    def _zero(off, size):
        pltpu.sync_copy(z128.at[pl.ds(0, size)], acc_sh.at[pl.ds(off, size)])
        pltpu.sync_copy(z16.at[pl.ds(0, size)], den_sh.at[pl.ds(off, size)])

    _striped_rows(s, _zero)
    pltpu.sync_copy(z16.at[pl.ds(0, CH)], sbufA)
    pltpu.sync_copy(z16.at[pl.ds(0, CH)], sbufB)
    plsc.subcore_barrier()

    per_sub = E // NSUB
    nchunks = per_sub // CH
    npairs = nchunks // 2
    base = s * per_sub
    iota = lax.iota(_i32, 16)
    zi = jnp.zeros((16,), _i32)
    oi = jnp.full((16,), 1, _i32)
    coff = c * N

    bufA = (edqA, srcA, dstA, src2A, arowA, browA, hrowA, sbufA, s2A,
            gsemA, isemA)
    bufB = (edqB, srcB, dstB, src2B, arowB, browB, hrowB, sbufB, s2B,
            gsemB, isemB)

    def issue(i, buf):
        edq, srcv, dstv, src2v, arow, brow, hrow, sbuf, s2v, gsem, isem = buf
        off = base + i * CH
        pltpu.make_async_copy(ed_h.at[pl.ds(off, CH)], edq, isem).wait()
        for j in range(CH // 16):
            r = 16 * j + iota
            sv_ = plsc.load_gather(edq, (r, zi))
            dv_ = plsc.load_gather(edq, (r, oi))
            srcv[pl.ds(16 * j, 16)] = sv_
            dstv[pl.ds(16 * j, 16)] = dv_
            src2v[pl.ds(16 * j, 16)] = sv_ + coff
        pltpu.async_copy(h2ab_h.at[src2v], hrow, gsem)
        pltpu.async_copy(a2_h.at[srcv], arow, gsem)
        pltpu.async_copy(a2_h.at[dstv], brow, gsem)

        @pl.when(i + 2 < nchunks)
        def _():
            pltpu.async_copy(ed_h.at[pl.ds(off + 2 * CH, CH)], edq, isem)

    def wait_g(buf):
        edq, srcv, dstv, src2v, arow, brow, hrow, sbuf, s2v, gsem, isem = buf
        pltpu.make_async_copy(h2ab_h.at[src2v], hrow, gsem).wait()
        pltpu.make_async_copy(a2_h.at[srcv], arow, gsem).wait()
        pltpu.make_async_copy(a2_h.at[dstv], brow, gsem).wait()

    def compute(buf):
        edq, srcv, dstv, src2v, arow, brow, hrow, sbuf, s2v, gsem, isem = buf
        for j in range(CH // 16):
            r = 16 * j + iota
            va = plsc.load_gather(arow, (r, zi))
            vb = plsc.load_gather(brow, (r, oi))
            xv = va + vb
            s16 = jnp.exp(jnp.maximum(xv, 0.2 * xv))
            plsc.store_scatter(sbuf, (r, zi), s16)
            s2v[pl.ds(16 * j, 16)] = s16

        def edge(j, cc):
            for u in range(4):
                jj = 4 * j + u
                sj = plsc.load_gather(s2v, (jnp.full((16,), jj, _i32),))
                for k in range(8):
                    hrow[jj, pl.ds(16 * k, 16)] = (
                        hrow[jj, pl.ds(16 * k, 16)] * sj)
            return cc

        lax.fori_loop(0, CH // 4, edge, 0)
        pltpu.sync_copy(sbuf, den_sh.at[dstv], add=True)
        pltpu.sync_copy(hrow, acc_sh.at[dstv], add=True)

    pltpu.async_copy(ed_h.at[pl.ds(base, CH)], edqA, isemA)
    pltpu.async_copy(ed_h.at[pl.ds(base + CH, CH)], edqB, isemB)
    issue(0, bufA)

    def pair(g, cc):
        i0 = 2 * g
        issue(i0 + 1, bufB)
        wait_g(bufA)
        compute(bufA)

        @pl.when(i0 + 2 < nchunks)
        def _():
            issue(i0 + 2, bufA)

        wait_g(bufB)
        compute(bufB)
        return cc

    lax.fori_loop(0, npairs, pair, 0)

    plsc.subcore_barrier()

    def _out0(off, size):
        pltpu.sync_copy(acc_sh.at[pl.ds(off, size)], houta.at[pl.ds(off, size)])
        pltpu.sync_copy(den_sh.at[pl.ds(off, size)], den2p.at[pl.ds(off, size)])

    def _out1(off, size):
        pltpu.sync_copy(acc_sh.at[pl.ds(off, size)], houtb.at[pl.ds(off, size)])

    @pl.when(c == 0)
    def _():
        _striped_rows(s, _out0)

    @pl.when(c == 1)
    def _():
        _striped_rows(s, _out1)

    half = nchunks // NCORE
    ahalf_pairs = half // 2
    setA = (edqA, srcA, dstA, arowA, browA, sbufA, s2A, gsemA, isemA)
    setB = (edqB, srcB, dstB, arowB, browB, sbufB, s2B, gsemB, isemB)

    def aissue(a, st):
        edq, srcv, dstv, arow, brow, drow, av, gsem, dsm = st
        off = base + (c * half + a) * CH
        pltpu.sync_copy(ed_h.at[pl.ds(off, CH)], edq)
        for j in range(CH // 16):
            r = 16 * j + iota
            srcv[pl.ds(16 * j, 16)] = plsc.load_gather(edq, (r, zi))
            dstv[pl.ds(16 * j, 16)] = plsc.load_gather(edq, (r, oi))
        pltpu.async_copy(a2_h.at[srcv], arow, gsem)
        pltpu.async_copy(a2_h.at[dstv], brow, gsem)
        pltpu.async_copy(den_sh.at[dstv], drow, dsm)

    def acompute(a, st):
        edq, srcv, dstv, arow, brow, drow, av, gsem, dsm = st
        pltpu.make_async_copy(a2_h.at[srcv], arow, gsem).wait()
        pltpu.make_async_copy(a2_h.at[dstv], brow, gsem).wait()
        pltpu.make_async_copy(den_sh.at[dstv], drow, dsm).wait()
        for j in range(CH // 16):
            r = 16 * j + iota
            va = plsc.load_gather(arow, (r, zi))
            vb = plsc.load_gather(brow, (r, oi))
            xv = va + vb
            s16 = jnp.exp(jnp.maximum(xv, 0.2 * xv))
            dv = plsc.load_gather(drow, (r, zi))
            av[pl.ds(16 * j, 16)] = s16 / (dv + 1e-16)
        off = base + (c * half + a) * CH
        pltpu.sync_copy(av, alpha_h.at[pl.ds(off, CH)])

    aissue(0, setA)

    def apair(g, cc):
        a0 = 2 * g
        aissue(a0 + 1, setB)
        acompute(a0, setA)

        @pl.when(a0 + 2 < half)
        def _():
            aissue(a0 + 2, setA)

        acompute(a0 + 1, setB)
        return cc

    lax.fori_loop(0, ahalf_pairs, apair, 0)
    if half % 2 == 1:
        acompute(half - 1, setA)


def _run_gat2(ed2, a2sd, h2ab, z128, z16):
    mesh = plsc.VectorSubcoreMesh(core_axis_name="c", subcore_axis_name="s",
                                  num_cores=NCORE, num_subcores=NSUB)
    buf = [
        pltpu.VMEM((CH, 2), _i32),
        pltpu.VMEM((CH,), _i32),
        pltpu.VMEM((CH,), _i32),
        pltpu.VMEM((CH,), _i32),
        pltpu.VMEM((CH, 16), _f32),
        pltpu.VMEM((CH, 16), _f32),
        pltpu.VMEM((CH, 128), _f32),
        pltpu.VMEM((CH, 16), _f32),
        pltpu.VMEM((CH,), _f32),
    ]
    fn = pl.kernel(
        _sc_gat2,
        out_type=[
            jax.ShapeDtypeStruct((N, 128), _f32),
            jax.ShapeDtypeStruct((N, 128), _f32),
            jax.ShapeDtypeStruct((N, 16), _f32),
            jax.ShapeDtypeStruct((E,), _f32),
        ],
        mesh=mesh,
        scratch_types=[
            pltpu.VMEM_SHARED((N, 128), _f32),
            pltpu.VMEM_SHARED((N, 16), _f32),
            *buf, *buf,
            pltpu.SemaphoreType.DMA,
            pltpu.SemaphoreType.DMA,
            pltpu.SemaphoreType.DMA,
            pltpu.SemaphoreType.DMA,
            pltpu.SemaphoreType.DMA,
        ],
        compiler_params=_SC_PARAMS,
    )
    return fn(ed2, a2sd, h2ab, z128, z16)


def _tc_head(ha_b, hb_b, den_b, bt_b, b2a, b2b, f1w, f1b, f2w, f2b,
             f3w, f3b, out_ref, pool_acc, cnt_acc):
    i = pl.program_id(0)

    @pl.when(i == 0)
    def _():
        pool_acc[...] = jnp.zeros_like(pool_acc)
        cnt_acc[...] = jnp.zeros_like(cnt_acc)

    den = den_b[...][:, :1]
    h2a = ha_b[...] / (den + 1e-16) + b2a[...]
    h2b = hb_b[...] / (den + 1e-16) + b2b[...]
    bt = bt_b[...].reshape(1, BN)
    g_iota = lax.broadcasted_iota(_i32, (G, BN), 0)
    oh = (g_iota == bt).astype(_f32)
    pool_acc[:, :128] += jnp.dot(oh, h2a, preferred_element_type=_f32)
    pool_acc[:, 128:] += jnp.dot(oh, h2b, preferred_element_type=_f32)
    cnt_acc[...] += jnp.dot(oh, jnp.ones((BN, 128), _f32),
                            preferred_element_type=_f32)

    @pl.when(i == (N // BN) - 1)
    def _():
        pooled = pool_acc[...] / jnp.maximum(cnt_acc[:, :1], 1.0)
        z = jnp.maximum(jnp.dot(pooled, f1w[...],
                                preferred_element_type=_f32) + f1b[...], 0.0)
        z = jnp.maximum(jnp.dot(z, f2w[...],
                                preferred_element_type=_f32) + f2b[...], 0.0)
        out_ref[...] = jnp.dot(z, f3w[...],
                               preferred_element_type=_f32) + f3b[...]


def _run_head(houta, houtb, den2p, batch3, b2a, b2b,
              fc1_w, fc1_b, fc2_w, fc2_b, fc3_w, fc3_b):
    return pl.pallas_call(
        _tc_head,
        grid=(N // BN,),
        in_specs=[
            pl.BlockSpec((BN, 128), lambda i: (i, 0)),
            pl.BlockSpec((BN, 128), lambda i: (i, 0)),
            pl.BlockSpec((BN, 16), lambda i: (i, 0)),
            pl.BlockSpec((1, 1, BN), lambda i: (i, 0, 0)),
            pl.BlockSpec((1, 128), lambda i: (0, 0)),
            pl.BlockSpec((1, 128), lambda i: (0, 0)),
            pl.BlockSpec((HID, 128), lambda i: (0, 0)),
            pl.BlockSpec((1, 128), lambda i: (0, 0)),
            pl.BlockSpec((128, 64), lambda i: (0, 0)),
            pl.BlockSpec((1, 64), lambda i: (0, 0)),
            pl.BlockSpec((64, NCLS), lambda i: (0, 0)),
            pl.BlockSpec((1, NCLS), lambda i: (0, 0)),
        ],
        out_specs=pl.BlockSpec((G, NCLS), lambda i: (0, 0)),
        out_shape=jax.ShapeDtypeStruct((G, NCLS), _f32),
        scratch_shapes=[
            pltpu.VMEM((G, HID), _f32),
            pltpu.VMEM((G, 128), _f32),
        ],
    )(houta, houtb, den2p, batch3, b2a, b2b,
      fc1_w, fc1_b, fc2_w, fc2_b, fc3_w, fc3_b)


def kernel(x, W1, a1_src, a1_dst, b1, W2, a2_src, a2_dst, b2,
           fc1_w, fc1_b, fc2_w, fc2_b, fc3_w, fc3_b, edge_index, batch):
    src = edge_index[0].astype(_i32)
    dst = edge_index[1].astype(_i32)

    hh = jnp.arange(64) // 8
    a1f_s = a1_src.reshape(64)
    a1f_d = a1_dst.reshape(64)
    eye8 = jnp.eye(8, dtype=_f32)
    ms_half = eye8[hh] * a1f_s[:, None]
    md_half = eye8[hh] * a1f_d[:, None]
    Ms = jnp.concatenate([ms_half, ms_half], axis=1)
    Md = jnp.concatenate([md_half, md_half], axis=1)
    E8 = jnp.repeat(eye8, 8, axis=1)
    A2m = jnp.zeros((HID, 16), _f32)
    A2m = A2m.at[:, 0].set(a2_src[0]).at[:, 1].set(a2_dst[0])

    z64 = jnp.zeros((RLAST, 64), _f32)
    z16 = jnp.zeros((RLAST, 16), _f32)
    z128 = jnp.zeros((RLAST, 128), _f32)

    ed2 = jnp.stack([src, dst], axis=1)
    hfeat, at, bt = _run_embed(x, W1, Ms, Md)
    acc0, acc1, den0, den1 = _run_gat1(ed2, at, bt, hfeat, z64, z16)
    h2ab3, a2sd = _run_mid(acc0, acc1, den0, den1,
                           b1.reshape(1, 64), E8, W2, A2m)
    h2ab = h2ab3.reshape(2 * N, 128)
    houta, houtb, den2p, alpha = _run_gat2(ed2, a2sd, h2ab, z128, z16)

    batch3 = batch.astype(_i32).reshape(N // BN, 1, BN)
    logits = _run_head(houta, houtb, den2p, batch3,
                       b2[:128].reshape(1, 128), b2[128:].reshape(1, 128),
                       fc1_w, fc1_b.reshape(1, 128),
                       fc2_w, fc2_b.reshape(1, 64),
                       fc3_w, fc3_b.reshape(1, NCLS))
    return logits, alpha.reshape(E, 1)
